# baseline (speedup 1.0000x reference)
import functools

import jax
import jax.numpy as jnp
from jax import lax
from jax.experimental import pallas as pl
from jax.experimental.pallas import tpu as pltpu
from jax.experimental.pallas import tpu_sc as plsc

N = 10000
NP = 10240
E = 160000
EP = 163840
H = 128
HEADS = 2
TN = 5
TE = 5
ETE = 5
EAE = 6
NEG = 0.2

f32 = jnp.float32
i32 = jnp.int32

_info = plsc.get_sparse_core_info()
NC = _info.num_cores
NS = _info.num_subcores
NW = NC * NS
EPW = EP // NW
CHA = 64
CHG = 128
NGA = CHA // 16

_MESH = dict(core_axis_name="c", subcore_axis_name="s")


def _wid():
    return lax.axis_index("s") * NC + lax.axis_index("c")


def _lky(v, slope):
    return jnp.where(v >= 0, v, slope * v)


@functools.partial(
    pl.kernel,
    out_type=[jax.ShapeDtypeStruct((2 * EP,), f32),
              jax.ShapeDtypeStruct((4 * NP,), f32)],
    mesh=plsc.VectorSubcoreMesh(**_MESH),
    compiler_params=pltpu.CompilerParams(needs_layout_passes=False),
    scratch_types=[pltpu.VMEM((CHA,), i32), pltpu.VMEM((CHA,), i32),
                   pltpu.VMEM((2 * CHA,), f32), pltpu.VMEM((2 * CHA,), f32),
                   pltpu.VMEM((2 * CHA,), i32), pltpu.VMEM((4 * NP,), f32),
                   pltpu.VMEM_SHARED((2 * NP,), f32)],
)
def _sc_attn(stab_h, c_h, src_h, dst_h, z2_h, ex_o, dp_o,
             dv, sv, cbuf, exbuf, idx2, stab_v, shared):
    cid = lax.axis_index("c")
    sid = lax.axis_index("s")
    stripe = 2 * NP // NS
    pltpu.sync_copy(stab_h, stab_v)
    pltpu.sync_copy(z2_h.at[pl.ds(sid * stripe, stripe)],
                    shared.at[pl.ds(sid * stripe, stripe)])
    plsc.subcore_barrier()
    base0 = _wid() * EPW
    iota = lax.iota(i32, 16)

    def body(i, _):
        b = base0 + i * CHA
        pltpu.sync_copy(dst_h.at[pl.ds(b, CHA)], dv)
        pltpu.sync_copy(src_h.at[pl.ds(b, CHA)], sv)
        pltpu.sync_copy(c_h.at[pl.ds(2 * b, 2 * CHA)], cbuf)
        for g in range(NGA):
            d16 = dv[pl.ds(g * 16, 16)]
            s16 = sv[pl.ds(g * 16, 16)]
            pos = 2 * (g * 16 + iota)
            sd0 = plsc.load_gather(stab_v, [d16 * 4])
            sd1 = plsc.load_gather(stab_v, [d16 * 4 + 1])
            ss0 = plsc.load_gather(stab_v, [s16 * 4 + 2])
            ss1 = plsc.load_gather(stab_v, [s16 * 4 + 3])
            c0 = plsc.load_gather(cbuf, [pos])
            c1 = plsc.load_gather(cbuf, [pos + 1])
            e0 = jnp.exp(_lky(sd0 + ss0 + c0, NEG))
            e1 = jnp.exp(_lky(sd1 + ss1 + c1, NEG))
            plsc.store_scatter(exbuf, [pos], e0)
            plsc.store_scatter(exbuf, [pos + 1], e1)
            plsc.store_scatter(idx2, [pos], d16 * 2)
            plsc.store_scatter(idx2, [pos + 1], d16 * 2 + 1)
        pltpu.sync_copy(exbuf, ex_o.at[pl.ds(2 * b, 2 * CHA)])
        pltpu.sync_copy(exbuf, shared.at[idx2], add=True)
        return 0

    lax.fori_loop(0, EPW // CHA, body, 0)
    plsc.subcore_barrier()
    pltpu.sync_copy(shared.at[pl.ds(sid * stripe, stripe)],
                    dp_o.at[pl.ds(cid * 2 * NP + sid * stripe, stripe)])


@functools.partial(
    pl.kernel,
    out_type=[jax.ShapeDtypeStruct((EP,), f32),
              jax.ShapeDtypeStruct((2 * 8 * NP,), f32)],
    mesh=plsc.VectorSubcoreMesh(**_MESH),
    compiler_params=pltpu.CompilerParams(needs_layout_passes=False),
    scratch_types=[pltpu.VMEM((CHA,), i32), pltpu.VMEM((2 * CHA,), f32),
                   pltpu.VMEM((6 * CHA,), f32), pltpu.VMEM((CHA,), f32),
                   pltpu.VMEM((CHA,), i32), pltpu.VMEM((CHA,), f32),
                   pltpu.VMEM((CHA,), f32), pltpu.VMEM((CHA,), f32),
                   pltpu.VMEM((CHA,), f32), pltpu.VMEM((CHA,), f32),
                   pltpu.VMEM((CHA,), f32), pltpu.VMEM((2 * NP,), f32),
                   pltpu.VMEM_SHARED((8 * NP,), f32)],
)
def _sc_w2(rtab_h, ex_h, eae_h, dst_h, zq_h, w_o, qp_o,
           dv, exl, eael, wbuf, qidx, q0, q1, q2, q3, q4, q5, rtab_v, qspm):
    cid = lax.axis_index("c")
    sid = lax.axis_index("s")
    stripe = 8 * NP // NS
    qb = [q0, q1, q2, q3, q4, q5]
    pltpu.sync_copy(rtab_h, rtab_v)
    pltpu.sync_copy(zq_h.at[pl.ds(sid * stripe, stripe)],
                    qspm.at[pl.ds(sid * stripe, stripe)])
    plsc.subcore_barrier()
    base0 = _wid() * EPW
    iota = lax.iota(i32, 16)

    def body(i, _):
        b = base0 + i * CHA
        pltpu.sync_copy(dst_h.at[pl.ds(b, CHA)], dv)
        pltpu.sync_copy(ex_h.at[pl.ds(2 * b, 2 * CHA)], exl)
        pltpu.sync_copy(eae_h.at[pl.ds(6 * b, 6 * CHA)], eael)
        for g in range(NGA):
            sl = pl.ds(g * 16, 16)
            d16 = dv[sl]
            loc = g * 16 + iota
            r0 = plsc.load_gather(rtab_v, [d16 * 2])
            r1 = plsc.load_gather(rtab_v, [d16 * 2 + 1])
            e0 = plsc.load_gather(exl, [loc * 2])
            e1 = plsc.load_gather(exl, [loc * 2 + 1])
            w16 = e0 * r0 + e1 * r1
            wbuf[sl] = w16
            qidx[sl] = d16 * 8
            for k in range(6):
                ek = plsc.load_gather(eael, [loc * 6 + k])
                qb[k][sl] = ek * w16
        pltpu.sync_copy(wbuf, w_o.at[pl.ds(b, CHA)])
        for k in range(6):
            pltpu.sync_copy(qb[k], qspm.at[qidx], add=True)
            if k < 5:
                for g in range(NGA):
                    sl = pl.ds(g * 16, 16)
                    qidx[sl] = qidx[sl] + 1
        return 0

    lax.fori_loop(0, EPW // CHA, body, 0)
    plsc.subcore_barrier()
    pltpu.sync_copy(qspm.at[pl.ds(sid * stripe, stripe)],
                    qp_o.at[pl.ds(cid * 8 * NP + sid * stripe, stripe)])


@functools.partial(
    pl.kernel,
    out_type=[jax.ShapeDtypeStruct((2 * NP, H), f32)],
    mesh=plsc.VectorSubcoreMesh(**_MESH),
    compiler_params=pltpu.CompilerParams(needs_layout_passes=False),
    scratch_types=[pltpu.VMEM((CHG,), i32), pltpu.VMEM((CHG,), i32),
                   pltpu.VMEM((CHG,), f32), pltpu.VMEM((CHG, H), f32),
                   pltpu.VMEM_SHARED((NP, H), f32), pltpu.SemaphoreType.DMA],
)
def _sc_fused(mnode_h, w_h, src_h, dst_h, z_h, out, sv, dv, wb, bg,
              shared, sem):
    cid = lax.axis_index("c")
    sid = lax.axis_index("s")
    stripe = NP // NS
    pltpu.sync_copy(z_h.at[pl.ds(sid * stripe, stripe)],
                    shared.at[pl.ds(sid * stripe, stripe)])
    plsc.subcore_barrier()
    base0 = _wid() * EPW
    iota = lax.iota(i32, 16)

    def body(i, _):
        b = base0 + i * CHG
        pltpu.sync_copy(src_h.at[pl.ds(b, CHG)], sv)
        pltpu.sync_copy(dst_h.at[pl.ds(b, CHG)], dv)
        pltpu.sync_copy(w_h.at[pl.ds(b, CHG)], wb)
        pltpu.async_copy(mnode_h.at[sv], bg, sem).wait()
        for g in range(CHG // 16):
            rows = g * 16 + iota
            w16 = wb[pl.ds(g * 16, 16)]
            for j in range(H):
                cj = jnp.full((16,), j, i32)
                col = plsc.load_gather(bg, [rows, cj])
                plsc.store_scatter(bg, [rows, cj], col * w16)
        pltpu.sync_copy(bg, shared.at[dv], add=True)
        return 0

    lax.fori_loop(0, EPW // CHG, body, 0)
    plsc.subcore_barrier()
    pltpu.sync_copy(shared.at[pl.ds(sid * stripe, stripe)],
                    out.at[pl.ds(cid * NP + sid * stripe, stripe)])


_NBLK = 1000
_EBLK = 2048


def _full(shape):
    return pl.BlockSpec(shape, lambda i: tuple(0 for _ in shape))


def _tc_h0(x, W, b):
    def body(xr, wr, br, out):
        out[...] = jax.nn.relu(
            jnp.dot(xr[...], wr[...], preferred_element_type=f32) + br[...])
    return pl.pallas_call(
        body,
        grid=(N // _NBLK,),
        in_specs=[pl.BlockSpec((_NBLK, H), lambda i: (i, 0)),
                  _full((H, H)), _full((1, H))],
        out_specs=pl.BlockSpec((_NBLK, H), lambda i: (i, 0)),
        out_shape=jax.ShapeDtypeStruct((N, H), f32),
    )(x, W, b)


def _tc_node(h, nt_r, hW, hb, aWd, aWs, lWn):
    def body(hr, ntr, hWr, hbr, aWdr, aWsr, lWnr, stab_o, mnode_o):
        hv = hr[...]
        nt = ntr[0, 0, :].reshape(_NBLK, 1)
        h2 = jnp.zeros((_NBLK, H), f32)
        for t in range(TN):
            proj = jnp.dot(hv, hWr[t], preferred_element_type=f32) \
                + hbr[t, :].reshape(1, H)
            h2 = h2 + jnp.where(nt == t, proj, 0.0)
        sd = jnp.dot(h2, aWdr[...], preferred_element_type=f32)
        ss = jnp.dot(h2, aWsr[...], preferred_element_type=f32)
        stab_o[...] = jnp.concatenate([sd, ss], axis=1)
        mnode_o[...] = jnp.dot(h2, lWnr[...], preferred_element_type=f32)

    return pl.pallas_call(
        body,
        grid=(N // _NBLK,),
        in_specs=[pl.BlockSpec((_NBLK, H), lambda i: (i, 0)),
                  pl.BlockSpec((1, 1, _NBLK), lambda i: (i, 0, 0)),
                  _full((TN, H, H)), _full((TN, H)),
                  _full((H, HEADS)), _full((H, HEADS)), _full((H, H))],
        out_specs=[pl.BlockSpec((_NBLK, 4), lambda i: (i, 0)),
                   pl.BlockSpec((_NBLK, H), lambda i: (i, 0))],
        out_shape=[jax.ShapeDtypeStruct((N, 4), f32),
                   jax.ShapeDtypeStruct((N, H), f32)],
    )(h, nt_r, hW, hb, aWd, aWs, lWn)


def _tc_edgec(et_r, ea, tab, aWe, aWa, eaW):
    def body(etr, ear, tabr, aWer, aWar, eaWr, c_o, eae_o):
        ctab = jnp.dot(_lky(tabr[...], 0.01), aWer[...],
                       preferred_element_type=f32)
        et = etr[0, 0, :].reshape(_EBLK, 1)
        iot = lax.broadcasted_iota(i32, (_EBLK, TE), 1)
        oh = jnp.where(et == iot, 1.0, 0.0).astype(f32)
        eae = _lky(jnp.dot(ear[...], eaWr[...], preferred_element_type=f32),
                   0.01)
        eae_o[...] = eae
        c_o[...] = jnp.dot(oh, ctab, preferred_element_type=f32) \
            + jnp.dot(eae, aWar[...], preferred_element_type=f32)

    return pl.pallas_call(
        body,
        grid=(EP // _EBLK,),
        in_specs=[pl.BlockSpec((1, 1, _EBLK), lambda i: (i, 0, 0)),
                  pl.BlockSpec((_EBLK, 2), lambda i: (i, 0)),
                  _full((TE, ETE)), _full((ETE, HEADS)),
                  _full((EAE, HEADS)), _full((2, EAE))],
        out_specs=[pl.BlockSpec((_EBLK, 2), lambda i: (i, 0)),
                   pl.BlockSpec((_EBLK, EAE), lambda i: (i, 0))],
        out_shape=[jax.ShapeDtypeStruct((EP, 2), f32),
                   jax.ShapeDtypeStruct((EP, EAE), f32)],
    )(et_r, ea, tab, aWe, aWa, eaW)


def _tc_rtab(dp):
    def body(dr, out):
        a = dr[...]
        out[...] = 0.5 / (a[0] + a[1] + 1e-16)
    return pl.pallas_call(
        body,
        grid=(1,),
        in_specs=[_full((2, NP, 2))],
        out_specs=_full((NP, 2)),
        out_shape=jax.ShapeDtypeStruct((NP, 2), f32),
    )(dp)


def _tc_hnext(apart, qpart, W6):
    def body(ar, qr, W6r, out):
        a = ar[...]
        q = qr[...]
        qs = (q[0] + q[1])[:, 0:EAE]
        out[...] = a[0] + a[1] + jnp.dot(qs, W6r[...],
                                         preferred_element_type=f32)
    return pl.pallas_call(
        body,
        grid=(N // _NBLK,),
        in_specs=[pl.BlockSpec((2, _NBLK, H), lambda i: (0, i, 0)),
                  pl.BlockSpec((2, _NBLK, 8), lambda i: (0, i, 0)),
                  _full((EAE, H))],
        out_specs=pl.BlockSpec((_NBLK, H), lambda i: (i, 0)),
        out_shape=jax.ShapeDtypeStruct((N, H), f32),
    )(apart, qpart, W6)


def _tc_outq(apart, qpart, W6, W, b):
    def body(ar, qr, W6r, wr, br, out):
        a = ar[...]
        q = qr[...]
        qs = (q[0] + q[1])[:, 0:EAE]
        hsum = a[0] + a[1] + jnp.dot(qs, W6r[...],
                                     preferred_element_type=f32)
        out[...] = jnp.dot(hsum, wr[...], preferred_element_type=f32) + br[...]
    return pl.pallas_call(
        body,
        grid=(N // _NBLK,),
        in_specs=[pl.BlockSpec((2, _NBLK, H), lambda i: (0, i, 0)),
                  pl.BlockSpec((2, _NBLK, 8), lambda i: (0, i, 0)),
                  _full((EAE, H)), _full((H, 64)), _full((1, 64))],
        out_specs=pl.BlockSpec((_NBLK, 64), lambda i: (i, 0)),
        out_shape=jax.ShapeDtypeStruct((N, 64), f32),
    )(apart, qpart, W6, W, b)


def kernel(x, edge_index, node_type, edge_type, edge_attr, lin_in_W, lin_in_b,
           hetero_W, hetero_b, edge_type_tab, edge_attr_W, att_W, lin_W,
           lin_out_W, lin_out_b):
    pad = EP - E
    src_p = jnp.concatenate([edge_index[0].astype(i32),
                             jnp.zeros((pad,), i32)])
    dst_p = jnp.concatenate([edge_index[1].astype(i32),
                             jnp.full((pad,), N, i32)])
    ea_p = jnp.concatenate([edge_attr, jnp.zeros((pad, 2), f32)])
    et_p = jnp.concatenate([edge_type.astype(i32), jnp.zeros((pad,), i32)])
    nt_r = node_type.astype(i32).reshape(N // _NBLK, 1, _NBLK)
    et_r = et_p.reshape(EP // _EBLK, 1, _EBLK)
    z2 = jnp.zeros((2 * NP,), f32)
    z128 = jnp.zeros((NP, H), f32)
    zq = jnp.zeros((8 * NP,), f32)
    zstab = jnp.zeros((NP - N, 4), f32)

    h = _tc_h0(x, lin_in_W, lin_in_b.reshape(1, H))
    for l in range(2):
        aW = att_W[l]
        stab, mnode = _tc_node(h, nt_r, hetero_W[l], hetero_b[l],
                               aW[0:H], aW[H:2 * H], lin_W[l][0:H])
        stab_f = jnp.concatenate([stab, zstab]).reshape(4 * NP)
        c, eae = _tc_edgec(et_r, ea_p, edge_type_tab[l],
                           aW[2 * H:2 * H + ETE], aW[2 * H + ETE:],
                           edge_attr_W[l])
        exf, dflat = _sc_attn(stab_f, c.reshape(2 * EP), src_p, dst_p, z2)
        rtab = _tc_rtab(dflat.reshape(2, NP, 2))
        wv, qflat = _sc_w2(rtab.reshape(2 * NP), exf, eae.reshape(6 * EP),
                           dst_p, zq)
        (apart,) = _sc_fused(mnode, wv, src_p, dst_p, z128)
        ap = apart.reshape(2, NP, H)
        qp = qflat.reshape(2, NP, 8)
        if l == 0:
            h = _tc_hnext(ap, qp, lin_W[l][H:])
    return _tc_outq(ap, qp, lin_W[1][H:], lin_out_W, lin_out_b.reshape(1, 64))


# speedup vs baseline: 1.8573x; 1.8573x over previous
import functools

import jax
import jax.numpy as jnp
from jax import lax
from jax.experimental import pallas as pl
from jax.experimental.pallas import tpu as pltpu
from jax.experimental.pallas import tpu_sc as plsc

N = 10000
NP = 10240
E = 160000
EP = 163840
H = 128
HEADS = 2
TN = 5
TE = 5
ETE = 5
EAE = 6
NEG = 0.2

f32 = jnp.float32
i32 = jnp.int32

_info = plsc.get_sparse_core_info()
NC = _info.num_cores
NS = _info.num_subcores
NW = NC * NS
EPW = EP // NW
CHA = 128
CHG = 128
NGA = CHA // 16

_MESH = dict(core_axis_name="c", subcore_axis_name="s")


def _wid():
    return lax.axis_index("s") * NC + lax.axis_index("c")


def _lky(v, slope):
    return jnp.where(v >= 0, v, slope * v)


@functools.partial(
    pl.kernel,
    out_type=[jax.ShapeDtypeStruct((2 * EP,), f32),
              jax.ShapeDtypeStruct((4 * NP,), f32)],
    mesh=plsc.VectorSubcoreMesh(**_MESH),
    compiler_params=pltpu.CompilerParams(needs_layout_passes=False),
    scratch_types=[pltpu.VMEM((CHA,), i32), pltpu.VMEM((CHA,), i32),
                   pltpu.VMEM((2 * CHA,), f32), pltpu.VMEM((2 * CHA,), f32),
                   pltpu.VMEM((CHA,), i32), pltpu.VMEM((CHA,), i32),
                   pltpu.VMEM((4 * NP,), f32),
                   pltpu.VMEM_SHARED((2 * NP,), f32)],
)
def _sc_attn(stab_h, c_h, src_h, dst_h, z2_h, ex_o, dp_o,
             dv, sv, cbuf, exbuf, idxa, idxb, stab_v, shared):
    cid = lax.axis_index("c")
    sid = lax.axis_index("s")
    stripe = 2 * NP // NS
    pltpu.sync_copy(stab_h, stab_v)
    pltpu.sync_copy(z2_h.at[pl.ds(sid * stripe, stripe)],
                    shared.at[pl.ds(sid * stripe, stripe)])
    plsc.subcore_barrier()
    base0 = _wid() * EPW
    iota = lax.iota(i32, 16)

    def body(i, _):
        b = base0 + i * CHA
        pltpu.sync_copy(dst_h.at[pl.ds(b, CHA)], dv)
        pltpu.sync_copy(src_h.at[pl.ds(b, CHA)], sv)
        pltpu.sync_copy(c_h.at[pl.ds(2 * b, 2 * CHA)], cbuf)
        for g in range(NGA):
            d16 = dv[pl.ds(g * 16, 16)]
            s16 = sv[pl.ds(g * 16, 16)]
            pos = 2 * (g * 16 + iota)
            sd0 = plsc.load_gather(stab_v, [d16 * 4])
            sd1 = plsc.load_gather(stab_v, [d16 * 4 + 1])
            ss0 = plsc.load_gather(stab_v, [s16 * 4 + 2])
            ss1 = plsc.load_gather(stab_v, [s16 * 4 + 3])
            c0 = plsc.load_gather(cbuf, [pos])
            c1 = plsc.load_gather(cbuf, [pos + 1])
            e0 = jnp.exp(_lky(sd0 + ss0 + c0, NEG))
            e1 = jnp.exp(_lky(sd1 + ss1 + c1, NEG))
            plsc.store_scatter(exbuf, [pos], e0)
            plsc.store_scatter(exbuf, [pos + 1], e1)
            ib = idxa if g < NGA // 2 else idxb
            ip = pos if g < NGA // 2 else pos - CHA
            plsc.store_scatter(ib, [ip], d16 * 2)
            plsc.store_scatter(ib, [ip + 1], d16 * 2 + 1)
        pltpu.sync_copy(exbuf, ex_o.at[pl.ds(2 * b, 2 * CHA)])
        pltpu.sync_copy(exbuf.at[pl.ds(0, CHA)], shared.at[idxa], add=True)
        pltpu.sync_copy(exbuf.at[pl.ds(CHA, CHA)], shared.at[idxb], add=True)
        return 0

    lax.fori_loop(0, EPW // CHA, body, 0)
    plsc.subcore_barrier()
    pltpu.sync_copy(shared.at[pl.ds(sid * stripe, stripe)],
                    dp_o.at[pl.ds(cid * 2 * NP + sid * stripe, stripe)])


@functools.partial(
    pl.kernel,
    out_type=[jax.ShapeDtypeStruct((EP,), f32),
              jax.ShapeDtypeStruct((EP, H), f32)],
    mesh=plsc.VectorSubcoreMesh(**_MESH),
    compiler_params=pltpu.CompilerParams(needs_layout_passes=False),
    scratch_types=[pltpu.VMEM((CHG,), i32), pltpu.VMEM((CHG,), i32),
                   pltpu.VMEM((2 * CHG,), f32), pltpu.VMEM((CHG,), f32),
                   pltpu.VMEM((2 * NP,), f32), pltpu.VMEM((CHG, H), f32),
                   pltpu.SemaphoreType.DMA],
)
def _sc_wg(rtab_h, ex_h, mnode_h, src_h, dst_h, w_o, g_o,
           sv, dv, exl, wbuf, rtab_v, bg, sem):
    pltpu.sync_copy(rtab_h, rtab_v)
    base0 = _wid() * EPW
    iota = lax.iota(i32, 16)

    def body(i, _):
        b = base0 + i * CHG
        pltpu.sync_copy(src_h.at[pl.ds(b, CHG)], sv)
        cp = pltpu.async_copy(mnode_h.at[sv], bg, sem)
        pltpu.sync_copy(dst_h.at[pl.ds(b, CHG)], dv)
        pltpu.sync_copy(ex_h.at[pl.ds(2 * b, 2 * CHG)], exl)
        for g in range(CHG // 16):
            d16 = dv[pl.ds(g * 16, 16)]
            pos = 2 * (g * 16 + iota)
            r0 = plsc.load_gather(rtab_v, [d16 * 2])
            r1 = plsc.load_gather(rtab_v, [d16 * 2 + 1])
            e0 = plsc.load_gather(exl, [pos])
            e1 = plsc.load_gather(exl, [pos + 1])
            wbuf[pl.ds(g * 16, 16)] = e0 * r0 + e1 * r1
        pltpu.sync_copy(wbuf, w_o.at[pl.ds(b, CHG)])
        cp.wait()
        pltpu.sync_copy(bg, g_o.at[pl.ds(b, CHG)])
        return 0

    lax.fori_loop(0, EPW // CHG, body, 0)


@functools.partial(
    pl.kernel,
    out_type=[jax.ShapeDtypeStruct((2 * NP, H), f32)],
    mesh=plsc.VectorSubcoreMesh(**_MESH),
    compiler_params=pltpu.CompilerParams(needs_layout_passes=False),
    scratch_types=[pltpu.VMEM((CHG,), i32), pltpu.VMEM((CHG, H), f32),
                   pltpu.VMEM_SHARED((NP, H), f32)],
)
def _sc_agg(wmsg_h, dst_h, z_h, out, dv, bw, shared):
    cid = lax.axis_index("c")
    sid = lax.axis_index("s")
    stripe = NP // NS
    pltpu.sync_copy(z_h.at[pl.ds(sid * stripe, stripe)],
                    shared.at[pl.ds(sid * stripe, stripe)])
    plsc.subcore_barrier()
    base0 = _wid() * EPW

    def body(i, _):
        b = base0 + i * CHG
        pltpu.sync_copy(dst_h.at[pl.ds(b, CHG)], dv)
        pltpu.sync_copy(wmsg_h.at[pl.ds(b, CHG)], bw)
        pltpu.sync_copy(bw, shared.at[dv], add=True)
        return 0

    lax.fori_loop(0, EPW // CHG, body, 0)
    plsc.subcore_barrier()
    pltpu.sync_copy(shared.at[pl.ds(sid * stripe, stripe)],
                    out.at[pl.ds(cid * NP + sid * stripe, stripe)])


_NBLK = 1000
_EBLK = 2048


def _full(shape):
    return pl.BlockSpec(shape, lambda i: tuple(0 for _ in shape))


def _tc_h0(x, W, b):
    def body(xr, wr, br, out):
        out[...] = jax.nn.relu(
            jnp.dot(xr[...], wr[...], preferred_element_type=f32) + br[...])
    return pl.pallas_call(
        body,
        grid=(N // _NBLK,),
        in_specs=[pl.BlockSpec((_NBLK, H), lambda i: (i, 0)),
                  _full((H, H)), _full((1, H))],
        out_specs=pl.BlockSpec((_NBLK, H), lambda i: (i, 0)),
        out_shape=jax.ShapeDtypeStruct((N, H), f32),
    )(x, W, b)


def _tc_node(h, nt_r, hW, hb, aWd, aWs, lWn):
    def body(hr, ntr, hWr, hbr, aWdr, aWsr, lWnr, stab_o, mnode_o):
        hv = hr[...]
        nt = ntr[0, 0, :].reshape(_NBLK, 1)
        h2 = jnp.zeros((_NBLK, H), f32)
        for t in range(TN):
            proj = jnp.dot(hv, hWr[t], preferred_element_type=f32) \
                + hbr[t, :].reshape(1, H)
            h2 = h2 + jnp.where(nt == t, proj, 0.0)
        sd = jnp.dot(h2, aWdr[...], preferred_element_type=f32)
        ss = jnp.dot(h2, aWsr[...], preferred_element_type=f32)
        stab_o[...] = jnp.concatenate([sd, ss], axis=1)
        mnode_o[...] = jnp.dot(h2, lWnr[...], preferred_element_type=f32)

    return pl.pallas_call(
        body,
        grid=(N // _NBLK,),
        in_specs=[pl.BlockSpec((_NBLK, H), lambda i: (i, 0)),
                  pl.BlockSpec((1, 1, _NBLK), lambda i: (i, 0, 0)),
                  _full((TN, H, H)), _full((TN, H)),
                  _full((H, HEADS)), _full((H, HEADS)), _full((H, H))],
        out_specs=[pl.BlockSpec((_NBLK, 4), lambda i: (i, 0)),
                   pl.BlockSpec((_NBLK, H), lambda i: (i, 0))],
        out_shape=[jax.ShapeDtypeStruct((N, 4), f32),
                   jax.ShapeDtypeStruct((N, H), f32)],
    )(h, nt_r, hW, hb, aWd, aWs, lWn)


def _tc_edgec(et_r, ea, tab, aWe, aWa, eaW):
    def body(etr, ear, tabr, aWer, aWar, eaWr, c_o):
        ctab = jnp.dot(_lky(tabr[...], 0.01), aWer[...],
                       preferred_element_type=f32)
        et = etr[0, 0, :].reshape(_EBLK, 1)
        iot = lax.broadcasted_iota(i32, (_EBLK, TE), 1)
        oh = jnp.where(et == iot, 1.0, 0.0).astype(f32)
        eae = _lky(jnp.dot(ear[...], eaWr[...], preferred_element_type=f32),
                   0.01)
        c_o[...] = jnp.dot(oh, ctab, preferred_element_type=f32) \
            + jnp.dot(eae, aWar[...], preferred_element_type=f32)

    return pl.pallas_call(
        body,
        grid=(EP // _EBLK,),
        in_specs=[pl.BlockSpec((1, 1, _EBLK), lambda i: (i, 0, 0)),
                  pl.BlockSpec((_EBLK, 2), lambda i: (i, 0)),
                  _full((TE, ETE)), _full((ETE, HEADS)),
                  _full((EAE, HEADS)), _full((2, EAE))],
        out_specs=pl.BlockSpec((_EBLK, 2), lambda i: (i, 0)),
        out_shape=jax.ShapeDtypeStruct((EP, 2), f32),
    )(et_r, ea, tab, aWe, aWa, eaW)


def _tc_rtab(dp):
    def body(dr, out):
        a = dr[...]
        out[...] = 0.5 / (a[0] + a[1] + 1e-16)
    return pl.pallas_call(
        body,
        grid=(1,),
        in_specs=[_full((2, NP, 2))],
        out_specs=_full((NP, 2)),
        out_shape=jax.ShapeDtypeStruct((NP, 2), f32),
    )(dp)


def _tc_wmsg(G, w, ea, eaW, W6):
    def body(Gr, wr, ear, eaWr, W6r, out):
        eae = _lky(jnp.dot(ear[...], eaWr[...], preferred_element_type=f32),
                   0.01)
        msg = Gr[...] + jnp.dot(eae, W6r[...], preferred_element_type=f32)
        out[...] = wr[...] * msg

    return pl.pallas_call(
        body,
        grid=(EP // _EBLK,),
        in_specs=[pl.BlockSpec((_EBLK, H), lambda i: (i, 0)),
                  pl.BlockSpec((_EBLK, 1), lambda i: (i, 0)),
                  pl.BlockSpec((_EBLK, 2), lambda i: (i, 0)),
                  _full((2, EAE)), _full((EAE, H))],
        out_specs=pl.BlockSpec((_EBLK, H), lambda i: (i, 0)),
        out_shape=jax.ShapeDtypeStruct((EP, H), f32),
    )(G, w, ea, eaW, W6)


def _tc_hsum(apart):
    def body(ar, out):
        a = ar[...]
        out[...] = a[0] + a[1]
    return pl.pallas_call(
        body,
        grid=(N // _NBLK,),
        in_specs=[pl.BlockSpec((2, _NBLK, H), lambda i: (0, i, 0))],
        out_specs=pl.BlockSpec((_NBLK, H), lambda i: (i, 0)),
        out_shape=jax.ShapeDtypeStruct((N, H), f32),
    )(apart)


def _tc_out(apart, W, b):
    def body(ar, wr, br, out):
        a = ar[...]
        out[...] = jnp.dot(a[0] + a[1], wr[...],
                           preferred_element_type=f32) + br[...]
    return pl.pallas_call(
        body,
        grid=(N // _NBLK,),
        in_specs=[pl.BlockSpec((2, _NBLK, H), lambda i: (0, i, 0)),
                  _full((H, 64)), _full((1, 64))],
        out_specs=pl.BlockSpec((_NBLK, 64), lambda i: (i, 0)),
        out_shape=jax.ShapeDtypeStruct((N, 64), f32),
    )(apart, W, b)


def kernel(x, edge_index, node_type, edge_type, edge_attr, lin_in_W, lin_in_b,
           hetero_W, hetero_b, edge_type_tab, edge_attr_W, att_W, lin_W,
           lin_out_W, lin_out_b):
    pad = EP - E
    src_p = jnp.concatenate([edge_index[0].astype(i32),
                             jnp.zeros((pad,), i32)])
    dst_p = jnp.concatenate([edge_index[1].astype(i32),
                             jnp.full((pad,), N, i32)])
    ea_p = jnp.concatenate([edge_attr, jnp.zeros((pad, 2), f32)])
    et_p = jnp.concatenate([edge_type.astype(i32), jnp.zeros((pad,), i32)])
    nt_r = node_type.astype(i32).reshape(N // _NBLK, 1, _NBLK)
    et_r = et_p.reshape(EP // _EBLK, 1, _EBLK)
    z2 = jnp.zeros((2 * NP,), f32)
    z128 = jnp.zeros((NP, H), f32)
    zstab = jnp.zeros((NP - N, 4), f32)

    h = _tc_h0(x, lin_in_W, lin_in_b.reshape(1, H))
    apart = None
    for l in range(2):
        aW = att_W[l]
        stab, mnode = _tc_node(h, nt_r, hetero_W[l], hetero_b[l],
                               aW[0:H], aW[H:2 * H], lin_W[l][0:H])
        stab_f = jnp.concatenate([stab, zstab]).reshape(4 * NP)
        c = _tc_edgec(et_r, ea_p, edge_type_tab[l],
                      aW[2 * H:2 * H + ETE], aW[2 * H + ETE:],
                      edge_attr_W[l])
        exf, dflat = _sc_attn(stab_f, c.reshape(2 * EP), src_p, dst_p, z2)
        rtab = _tc_rtab(dflat.reshape(2, NP, 2))
        wv, G = _sc_wg(rtab.reshape(2 * NP), exf, mnode, src_p, dst_p)
        wmsg = _tc_wmsg(G, wv.reshape(EP, 1), ea_p, edge_attr_W[l],
                        lin_W[l][H:])
        (apart,) = _sc_agg(wmsg, dst_p, z128)
        if l == 0:
            h = _tc_hsum(apart.reshape(2, NP, H))
    return _tc_out(apart.reshape(2, NP, H), lin_out_W, lin_out_b.reshape(1, 64))


# speedup vs baseline: 1.9190x; 1.0332x over previous
import functools

import jax
import jax.numpy as jnp
from jax import lax
from jax.experimental import pallas as pl
from jax.experimental.pallas import tpu as pltpu
from jax.experimental.pallas import tpu_sc as plsc

N = 10000
NP = 10240
E = 160000
EP = 163840
H = 128
HEADS = 2
TN = 5
TE = 5
ETE = 5
EAE = 6
NEG = 0.2

f32 = jnp.float32
i32 = jnp.int32

_info = plsc.get_sparse_core_info()
NC = _info.num_cores
NS = _info.num_subcores
NW = NC * NS
EPW = EP // NW
CHA = 128
CHG = 128
NGA = CHA // 16

_MESH = dict(core_axis_name="c", subcore_axis_name="s")


def _wid():
    return lax.axis_index("s") * NC + lax.axis_index("c")


def _lky(v, slope):
    return jnp.where(v >= 0, v, slope * v)


@functools.partial(
    pl.kernel,
    out_type=[jax.ShapeDtypeStruct((2 * EP,), f32),
              jax.ShapeDtypeStruct((4 * NP,), f32)],
    mesh=plsc.VectorSubcoreMesh(**_MESH),
    compiler_params=pltpu.CompilerParams(needs_layout_passes=False),
    scratch_types=[pltpu.VMEM((CHA,), i32), pltpu.VMEM((CHA,), i32),
                   pltpu.VMEM((2 * CHA,), f32), pltpu.VMEM((2 * CHA,), f32),
                   pltpu.VMEM((CHA,), i32), pltpu.VMEM((CHA,), i32),
                   pltpu.VMEM((4 * NP,), f32),
                   pltpu.VMEM_SHARED((2 * NP,), f32)],
)
def _sc_attn(stab_h, c_h, src_h, dst_h, z2_h, ex_o, dp_o,
             dv, sv, cbuf, exbuf, idxa, idxb, stab_v, shared):
    cid = lax.axis_index("c")
    sid = lax.axis_index("s")
    stripe = 2 * NP // NS
    pltpu.sync_copy(stab_h, stab_v)
    pltpu.sync_copy(z2_h.at[pl.ds(sid * stripe, stripe)],
                    shared.at[pl.ds(sid * stripe, stripe)])
    plsc.subcore_barrier()
    base0 = _wid() * EPW
    iota = lax.iota(i32, 16)

    def body(i, _):
        b = base0 + i * CHA
        pltpu.sync_copy(dst_h.at[pl.ds(b, CHA)], dv)
        pltpu.sync_copy(src_h.at[pl.ds(b, CHA)], sv)
        pltpu.sync_copy(c_h.at[pl.ds(2 * b, 2 * CHA)], cbuf)
        for g in range(NGA):
            d16 = dv[pl.ds(g * 16, 16)]
            s16 = sv[pl.ds(g * 16, 16)]
            pos = 2 * (g * 16 + iota)
            sd0 = plsc.load_gather(stab_v, [d16 * 4])
            sd1 = plsc.load_gather(stab_v, [d16 * 4 + 1])
            ss0 = plsc.load_gather(stab_v, [s16 * 4 + 2])
            ss1 = plsc.load_gather(stab_v, [s16 * 4 + 3])
            c0 = plsc.load_gather(cbuf, [pos])
            c1 = plsc.load_gather(cbuf, [pos + 1])
            e0 = jnp.exp(_lky(sd0 + ss0 + c0, NEG))
            e1 = jnp.exp(_lky(sd1 + ss1 + c1, NEG))
            plsc.store_scatter(exbuf, [pos], e0)
            plsc.store_scatter(exbuf, [pos + 1], e1)
            ib = idxa if g < NGA // 2 else idxb
            ip = pos if g < NGA // 2 else pos - CHA
            plsc.store_scatter(ib, [ip], d16 * 2)
            plsc.store_scatter(ib, [ip + 1], d16 * 2 + 1)
        pltpu.sync_copy(exbuf, ex_o.at[pl.ds(2 * b, 2 * CHA)])
        pltpu.sync_copy(exbuf.at[pl.ds(0, CHA)], shared.at[idxa], add=True)
        pltpu.sync_copy(exbuf.at[pl.ds(CHA, CHA)], shared.at[idxb], add=True)
        return 0

    lax.fori_loop(0, EPW // CHA, body, 0)
    plsc.subcore_barrier()
    pltpu.sync_copy(shared.at[pl.ds(sid * stripe, stripe)],
                    dp_o.at[pl.ds(cid * 2 * NP + sid * stripe, stripe)])


@functools.partial(
    pl.kernel,
    out_type=[jax.ShapeDtypeStruct((EP,), f32),
              jax.ShapeDtypeStruct((EP, H), f32)],
    mesh=plsc.VectorSubcoreMesh(**_MESH),
    compiler_params=pltpu.CompilerParams(needs_layout_passes=False),
    scratch_types=[pltpu.VMEM((CHG,), i32), pltpu.VMEM((CHG,), i32),
                   pltpu.VMEM((2 * CHG,), f32), pltpu.VMEM((CHG,), f32),
                   pltpu.VMEM((4 * NP,), f32),
                   pltpu.VMEM((2 * NP,), f32), pltpu.VMEM((CHG, H), f32),
                   pltpu.SemaphoreType.DMA],
)
def _sc_wg(dp_h, ex_h, mnode_h, src_h, dst_h, w_o, g_o,
           sv, dv, exl, wbuf, dpv, rtab_v, bg, sem):
    pltpu.sync_copy(dp_h, dpv)

    def rbody(j, _):
        sl = pl.ds(j * 16, 16)
        sl2 = pl.ds(2 * NP + j * 16, 16)
        rtab_v[sl] = 0.5 / (dpv[sl] + dpv[sl2] + 1e-16)
        return 0

    lax.fori_loop(0, 2 * NP // 16, rbody, 0)
    base0 = _wid() * EPW
    iota = lax.iota(i32, 16)

    def body(i, _):
        b = base0 + i * CHG
        pltpu.sync_copy(src_h.at[pl.ds(b, CHG)], sv)
        cp = pltpu.async_copy(mnode_h.at[sv], bg, sem)
        pltpu.sync_copy(dst_h.at[pl.ds(b, CHG)], dv)
        pltpu.sync_copy(ex_h.at[pl.ds(2 * b, 2 * CHG)], exl)
        for g in range(CHG // 16):
            d16 = dv[pl.ds(g * 16, 16)]
            pos = 2 * (g * 16 + iota)
            r0 = plsc.load_gather(rtab_v, [d16 * 2])
            r1 = plsc.load_gather(rtab_v, [d16 * 2 + 1])
            e0 = plsc.load_gather(exl, [pos])
            e1 = plsc.load_gather(exl, [pos + 1])
            wbuf[pl.ds(g * 16, 16)] = e0 * r0 + e1 * r1
        pltpu.sync_copy(wbuf, w_o.at[pl.ds(b, CHG)])
        cp.wait()
        pltpu.sync_copy(bg, g_o.at[pl.ds(b, CHG)])
        return 0

    lax.fori_loop(0, EPW // CHG, body, 0)


@functools.partial(
    pl.kernel,
    out_type=[jax.ShapeDtypeStruct((2 * NP, H), f32)],
    mesh=plsc.VectorSubcoreMesh(**_MESH),
    compiler_params=pltpu.CompilerParams(needs_layout_passes=False),
    scratch_types=[pltpu.VMEM((CHG,), i32), pltpu.VMEM((CHG,), i32),
                   pltpu.VMEM((CHG, H), f32), pltpu.VMEM((CHG, H), f32),
                   pltpu.VMEM_SHARED((NP, H), f32),
                   pltpu.SemaphoreType.DMA, pltpu.SemaphoreType.DMA],
)
def _sc_agg(wmsg_h, dst_h, z_h, out, dv0, dv1, bw0, bw1, shared,
            sem0, sem1):
    cid = lax.axis_index("c")
    sid = lax.axis_index("s")
    stripe = NP // NS
    nch = EPW // CHG
    dvs = [dv0, dv1]
    bws = [bw0, bw1]
    sems = [sem0, sem1]
    pltpu.sync_copy(z_h.at[pl.ds(sid * stripe, stripe)],
                    shared.at[pl.ds(sid * stripe, stripe)])
    plsc.subcore_barrier()
    base0 = _wid() * EPW
    for k in range(2):
        b = base0 + k * CHG
        pltpu.sync_copy(dst_h.at[pl.ds(b, CHG)], dvs[k])
        pltpu.async_copy(wmsg_h.at[pl.ds(b, CHG)], bws[k], sems[k])

    def body(i, _):
        for k in range(2):
            ch = 2 * i + k
            b = base0 + ch * CHG
            pltpu.make_async_copy(wmsg_h.at[pl.ds(b, CHG)], bws[k],
                                  sems[k]).wait()
            pltpu.sync_copy(bws[k], shared.at[dvs[k]], add=True)

            @pl.when(ch + 2 < nch)
            def _():
                b2 = base0 + (ch + 2) * CHG
                pltpu.sync_copy(dst_h.at[pl.ds(b2, CHG)], dvs[k])
                pltpu.async_copy(wmsg_h.at[pl.ds(b2, CHG)], bws[k], sems[k])
        return 0

    lax.fori_loop(0, nch // 2, body, 0)
    plsc.subcore_barrier()
    pltpu.sync_copy(shared.at[pl.ds(sid * stripe, stripe)],
                    out.at[pl.ds(cid * NP + sid * stripe, stripe)])


_NBLK = 1000
_EBLK = 2048


def _full(shape):
    return pl.BlockSpec(shape, lambda i: tuple(0 for _ in shape))


def _tc_h0(x, W, b):
    def body(xr, wr, br, out):
        out[...] = jax.nn.relu(
            jnp.dot(xr[...], wr[...], preferred_element_type=f32) + br[...])
    return pl.pallas_call(
        body,
        grid=(N // _NBLK,),
        in_specs=[pl.BlockSpec((_NBLK, H), lambda i: (i, 0)),
                  _full((H, H)), _full((1, H))],
        out_specs=pl.BlockSpec((_NBLK, H), lambda i: (i, 0)),
        out_shape=jax.ShapeDtypeStruct((N, H), f32),
    )(x, W, b)


def _tc_node(h, nt_r, hW, hb, aWd, aWs, lWn):
    def body(hr, ntr, hWr, hbr, aWdr, aWsr, lWnr, stab_o, mnode_o):
        hv = hr[...]
        nt = ntr[0, 0, :].reshape(_NBLK, 1)
        h2 = jnp.zeros((_NBLK, H), f32)
        for t in range(TN):
            proj = jnp.dot(hv, hWr[t], preferred_element_type=f32) \
                + hbr[t, :].reshape(1, H)
            h2 = h2 + jnp.where(nt == t, proj, 0.0)
        sd = jnp.dot(h2, aWdr[...], preferred_element_type=f32)
        ss = jnp.dot(h2, aWsr[...], preferred_element_type=f32)
        stab_o[...] = jnp.concatenate([sd, ss], axis=1)
        mnode_o[...] = jnp.dot(h2, lWnr[...], preferred_element_type=f32)

    return pl.pallas_call(
        body,
        grid=(N // _NBLK,),
        in_specs=[pl.BlockSpec((_NBLK, H), lambda i: (i, 0)),
                  pl.BlockSpec((1, 1, _NBLK), lambda i: (i, 0, 0)),
                  _full((TN, H, H)), _full((TN, H)),
                  _full((H, HEADS)), _full((H, HEADS)), _full((H, H))],
        out_specs=[pl.BlockSpec((_NBLK, 4), lambda i: (i, 0)),
                   pl.BlockSpec((_NBLK, H), lambda i: (i, 0))],
        out_shape=[jax.ShapeDtypeStruct((N, 4), f32),
                   jax.ShapeDtypeStruct((N, H), f32)],
    )(h, nt_r, hW, hb, aWd, aWs, lWn)


def _tc_edgec(et_r, ea, tab, aWe, aWa, eaW):
    def body(etr, ear, tabr, aWer, aWar, eaWr, c_o):
        ctab = jnp.dot(_lky(tabr[...], 0.01), aWer[...],
                       preferred_element_type=f32)
        et = etr[0, 0, :].reshape(_EBLK, 1)
        iot = lax.broadcasted_iota(i32, (_EBLK, TE), 1)
        oh = jnp.where(et == iot, 1.0, 0.0).astype(f32)
        eae = _lky(jnp.dot(ear[...], eaWr[...], preferred_element_type=f32),
                   0.01)
        c_o[...] = jnp.dot(oh, ctab, preferred_element_type=f32) \
            + jnp.dot(eae, aWar[...], preferred_element_type=f32)

    return pl.pallas_call(
        body,
        grid=(EP // _EBLK,),
        in_specs=[pl.BlockSpec((1, 1, _EBLK), lambda i: (i, 0, 0)),
                  pl.BlockSpec((_EBLK, 2), lambda i: (i, 0)),
                  _full((TE, ETE)), _full((ETE, HEADS)),
                  _full((EAE, HEADS)), _full((2, EAE))],
        out_specs=pl.BlockSpec((_EBLK, 2), lambda i: (i, 0)),
        out_shape=jax.ShapeDtypeStruct((EP, 2), f32),
    )(et_r, ea, tab, aWe, aWa, eaW)


def _tc_rtab(dp):
    def body(dr, out):
        a = dr[...]
        out[...] = 0.5 / (a[0] + a[1] + 1e-16)
    return pl.pallas_call(
        body,
        grid=(1,),
        in_specs=[_full((2, NP, 2))],
        out_specs=_full((NP, 2)),
        out_shape=jax.ShapeDtypeStruct((NP, 2), f32),
    )(dp)


def _tc_wmsg(G, w, ea, eaW, W6):
    def body(Gr, wr, ear, eaWr, W6r, out):
        eae = _lky(jnp.dot(ear[...], eaWr[...], preferred_element_type=f32),
                   0.01)
        msg = Gr[...] + jnp.dot(eae, W6r[...], preferred_element_type=f32)
        out[...] = wr[...] * msg

    return pl.pallas_call(
        body,
        grid=(EP // _EBLK,),
        in_specs=[pl.BlockSpec((_EBLK, H), lambda i: (i, 0)),
                  pl.BlockSpec((_EBLK, 1), lambda i: (i, 0)),
                  pl.BlockSpec((_EBLK, 2), lambda i: (i, 0)),
                  _full((2, EAE)), _full((EAE, H))],
        out_specs=pl.BlockSpec((_EBLK, H), lambda i: (i, 0)),
        out_shape=jax.ShapeDtypeStruct((EP, H), f32),
    )(G, w, ea, eaW, W6)


def _tc_hsum(apart):
    def body(ar, out):
        a = ar[...]
        out[...] = a[0] + a[1]
    return pl.pallas_call(
        body,
        grid=(N // _NBLK,),
        in_specs=[pl.BlockSpec((2, _NBLK, H), lambda i: (0, i, 0))],
        out_specs=pl.BlockSpec((_NBLK, H), lambda i: (i, 0)),
        out_shape=jax.ShapeDtypeStruct((N, H), f32),
    )(apart)


def _tc_out(apart, W, b):
    def body(ar, wr, br, out):
        a = ar[...]
        out[...] = jnp.dot(a[0] + a[1], wr[...],
                           preferred_element_type=f32) + br[...]
    return pl.pallas_call(
        body,
        grid=(N // _NBLK,),
        in_specs=[pl.BlockSpec((2, _NBLK, H), lambda i: (0, i, 0)),
                  _full((H, 64)), _full((1, 64))],
        out_specs=pl.BlockSpec((_NBLK, 64), lambda i: (i, 0)),
        out_shape=jax.ShapeDtypeStruct((N, 64), f32),
    )(apart, W, b)


def kernel(x, edge_index, node_type, edge_type, edge_attr, lin_in_W, lin_in_b,
           hetero_W, hetero_b, edge_type_tab, edge_attr_W, att_W, lin_W,
           lin_out_W, lin_out_b):
    pad = EP - E
    src_p = jnp.concatenate([edge_index[0].astype(i32),
                             jnp.zeros((pad,), i32)])
    dst_p = jnp.concatenate([edge_index[1].astype(i32),
                             jnp.full((pad,), N, i32)])
    ea_p = jnp.concatenate([edge_attr, jnp.zeros((pad, 2), f32)])
    et_p = jnp.concatenate([edge_type.astype(i32), jnp.zeros((pad,), i32)])
    nt_r = node_type.astype(i32).reshape(N // _NBLK, 1, _NBLK)
    et_r = et_p.reshape(EP // _EBLK, 1, _EBLK)
    z2 = jnp.zeros((2 * NP,), f32)
    z128 = jnp.zeros((NP, H), f32)
    zstab = jnp.zeros((NP - N, 4), f32)

    h = _tc_h0(x, lin_in_W, lin_in_b.reshape(1, H))
    apart = None
    for l in range(2):
        aW = att_W[l]
        stab, mnode = _tc_node(h, nt_r, hetero_W[l], hetero_b[l],
                               aW[0:H], aW[H:2 * H], lin_W[l][0:H])
        stab_f = jnp.concatenate([stab, zstab]).reshape(4 * NP)
        c = _tc_edgec(et_r, ea_p, edge_type_tab[l],
                      aW[2 * H:2 * H + ETE], aW[2 * H + ETE:],
                      edge_attr_W[l])
        exf, dflat = _sc_attn(stab_f, c.reshape(2 * EP), src_p, dst_p, z2)
        wv, G = _sc_wg(dflat, exf, mnode, src_p, dst_p)
        wmsg = _tc_wmsg(G, wv.reshape(EP, 1), ea_p, edge_attr_W[l],
                        lin_W[l][H:])
        (apart,) = _sc_agg(wmsg, dst_p, z128)
        if l == 0:
            h = _tc_hsum(apart.reshape(2, NP, H))
    return _tc_out(apart.reshape(2, NP, H), lin_out_W, lin_out_b.reshape(1, 64))


# speedup vs baseline: 2.0209x; 1.0531x over previous
import functools

import jax
import jax.numpy as jnp
from jax import lax
from jax.experimental import pallas as pl
from jax.experimental.pallas import tpu as pltpu
from jax.experimental.pallas import tpu_sc as plsc

N = 10000
NP = 10240
E = 160000
EP = 163840
H = 128
HEADS = 2
TN = 5
TE = 5
ETE = 5
EAE = 6
NEG = 0.2

f32 = jnp.float32
i32 = jnp.int32

_info = plsc.get_sparse_core_info()
NC = _info.num_cores
NS = _info.num_subcores
NW = NC * NS
EPW = EP // NW
CHA = 128
CHG = 128
NGA = CHA // 16

_MESH = dict(core_axis_name="c", subcore_axis_name="s")


def _wid():
    return lax.axis_index("s") * NC + lax.axis_index("c")


def _lky(v, slope):
    return jnp.where(v >= 0, v, slope * v)


@functools.partial(
    pl.kernel,
    out_type=[jax.ShapeDtypeStruct((2 * EP,), f32),
              jax.ShapeDtypeStruct((4 * NP,), f32)],
    mesh=plsc.VectorSubcoreMesh(**_MESH),
    compiler_params=pltpu.CompilerParams(needs_layout_passes=False),
    scratch_types=[pltpu.VMEM((CHA,), i32), pltpu.VMEM((CHA,), i32),
                   pltpu.VMEM((2 * CHA,), f32), pltpu.VMEM((2 * CHA,), f32),
                   pltpu.VMEM((CHA,), i32), pltpu.VMEM((CHA,), i32),
                   pltpu.VMEM((4 * NP,), f32),
                   pltpu.VMEM_SHARED((2 * NP,), f32)],
)
def _sc_attn(stab_h, c_h, src_h, dst_h, z2_h, ex_o, dp_o,
             dv, sv, cbuf, exbuf, idxa, idxb, stab_v, shared):
    cid = lax.axis_index("c")
    sid = lax.axis_index("s")
    stripe = 2 * NP // NS
    pltpu.sync_copy(stab_h, stab_v)
    pltpu.sync_copy(z2_h.at[pl.ds(sid * stripe, stripe)],
                    shared.at[pl.ds(sid * stripe, stripe)])
    plsc.subcore_barrier()
    base0 = _wid() * EPW
    iota = lax.iota(i32, 16)

    def body(i, _):
        b = base0 + i * CHA
        pltpu.sync_copy(dst_h.at[pl.ds(b, CHA)], dv)
        pltpu.sync_copy(src_h.at[pl.ds(b, CHA)], sv)
        pltpu.sync_copy(c_h.at[pl.ds(2 * b, 2 * CHA)], cbuf)
        for g in range(NGA):
            d16 = dv[pl.ds(g * 16, 16)]
            s16 = sv[pl.ds(g * 16, 16)]
            pos = 2 * (g * 16 + iota)
            sd0 = plsc.load_gather(stab_v, [d16 * 4])
            sd1 = plsc.load_gather(stab_v, [d16 * 4 + 1])
            ss0 = plsc.load_gather(stab_v, [s16 * 4 + 2])
            ss1 = plsc.load_gather(stab_v, [s16 * 4 + 3])
            c0 = plsc.load_gather(cbuf, [pos])
            c1 = plsc.load_gather(cbuf, [pos + 1])
            e0 = jnp.exp(_lky(sd0 + ss0 + c0, NEG))
            e1 = jnp.exp(_lky(sd1 + ss1 + c1, NEG))
            plsc.store_scatter(exbuf, [pos], e0)
            plsc.store_scatter(exbuf, [pos + 1], e1)
            ib = idxa if g < NGA // 2 else idxb
            ip = pos if g < NGA // 2 else pos - CHA
            plsc.store_scatter(ib, [ip], d16 * 2)
            plsc.store_scatter(ib, [ip + 1], d16 * 2 + 1)
        pltpu.sync_copy(exbuf, ex_o.at[pl.ds(2 * b, 2 * CHA)])
        pltpu.sync_copy(exbuf.at[pl.ds(0, CHA)], shared.at[idxa], add=True)
        pltpu.sync_copy(exbuf.at[pl.ds(CHA, CHA)], shared.at[idxb], add=True)
        return 0

    lax.fori_loop(0, EPW // CHA, body, 0)
    plsc.subcore_barrier()
    pltpu.sync_copy(shared.at[pl.ds(sid * stripe, stripe)],
                    dp_o.at[pl.ds(cid * 2 * NP + sid * stripe, stripe)])


@functools.partial(
    pl.kernel,
    out_type=[jax.ShapeDtypeStruct((EP,), f32),
              jax.ShapeDtypeStruct((EP, H), f32)],
    mesh=plsc.VectorSubcoreMesh(**_MESH),
    compiler_params=pltpu.CompilerParams(needs_layout_passes=False),
    scratch_types=[pltpu.VMEM((CHG,), i32), pltpu.VMEM((CHG,), i32),
                   pltpu.VMEM((CHG,), i32),
                   pltpu.VMEM((2 * CHG,), f32), pltpu.VMEM((CHG,), f32),
                   pltpu.VMEM((4 * NP,), f32),
                   pltpu.VMEM((2 * NP,), f32), pltpu.VMEM((CHG, H), f32),
                   pltpu.VMEM((CHG, H), f32),
                   pltpu.SemaphoreType.DMA, pltpu.SemaphoreType.DMA],
)
def _sc_wg(dp_h, ex_h, mnode_h, src_h, dst_h, w_o, g_o,
           sv0, sv1, dv, exl, wbuf, dpv, rtab_v, bg0, bg1, sem0, sem1):
    pltpu.sync_copy(dp_h, dpv)

    def rbody(j, _):
        sl = pl.ds(j * 16, 16)
        sl2 = pl.ds(2 * NP + j * 16, 16)
        rtab_v[sl] = 0.5 / (dpv[sl] + dpv[sl2] + 1e-16)
        return 0

    lax.fori_loop(0, 2 * NP // 16, rbody, 0)
    base0 = _wid() * EPW
    iota = lax.iota(i32, 16)
    nch = EPW // CHG
    svs = [sv0, sv1]
    bgs = [bg0, bg1]
    sems = [sem0, sem1]
    for k in range(2):
        b = base0 + k * CHG
        pltpu.sync_copy(src_h.at[pl.ds(b, CHG)], svs[k])
        pltpu.async_copy(mnode_h.at[svs[k]], bgs[k], sems[k])

    def body(i, _):
        for k in range(2):
            ch = 2 * i + k
            b = base0 + ch * CHG
            pltpu.sync_copy(dst_h.at[pl.ds(b, CHG)], dv)
            pltpu.sync_copy(ex_h.at[pl.ds(2 * b, 2 * CHG)], exl)
            for g in range(CHG // 16):
                d16 = dv[pl.ds(g * 16, 16)]
                pos = 2 * (g * 16 + iota)
                r0 = plsc.load_gather(rtab_v, [d16 * 2])
                r1 = plsc.load_gather(rtab_v, [d16 * 2 + 1])
                e0 = plsc.load_gather(exl, [pos])
                e1 = plsc.load_gather(exl, [pos + 1])
                wbuf[pl.ds(g * 16, 16)] = e0 * r0 + e1 * r1
            pltpu.sync_copy(wbuf, w_o.at[pl.ds(b, CHG)])
            pltpu.make_async_copy(mnode_h.at[svs[k]], bgs[k], sems[k]).wait()
            pltpu.sync_copy(bgs[k], g_o.at[pl.ds(b, CHG)])

            @pl.when(ch + 2 < nch)
            def _():
                b2 = base0 + (ch + 2) * CHG
                pltpu.sync_copy(src_h.at[pl.ds(b2, CHG)], svs[k])
                pltpu.async_copy(mnode_h.at[svs[k]], bgs[k], sems[k])
        return 0

    lax.fori_loop(0, nch // 2, body, 0)


@functools.partial(
    pl.kernel,
    out_type=[jax.ShapeDtypeStruct((2 * NP, H), f32)],
    mesh=plsc.VectorSubcoreMesh(**_MESH),
    compiler_params=pltpu.CompilerParams(needs_layout_passes=False),
    scratch_types=[pltpu.VMEM((CHG,), i32), pltpu.VMEM((CHG,), i32),
                   pltpu.VMEM((CHG, H), f32), pltpu.VMEM((CHG, H), f32),
                   pltpu.VMEM_SHARED((NP, H), f32),
                   pltpu.SemaphoreType.DMA, pltpu.SemaphoreType.DMA],
)
def _sc_agg(wmsg_h, dst_h, z_h, out, dv0, dv1, bw0, bw1, shared,
            sem0, sem1):
    cid = lax.axis_index("c")
    sid = lax.axis_index("s")
    stripe = NP // NS
    nch = EPW // CHG
    dvs = [dv0, dv1]
    bws = [bw0, bw1]
    sems = [sem0, sem1]
    pltpu.sync_copy(z_h.at[pl.ds(sid * stripe, stripe)],
                    shared.at[pl.ds(sid * stripe, stripe)])
    plsc.subcore_barrier()
    base0 = _wid() * EPW
    for k in range(2):
        b = base0 + k * CHG
        pltpu.sync_copy(dst_h.at[pl.ds(b, CHG)], dvs[k])
        pltpu.async_copy(wmsg_h.at[pl.ds(b, CHG)], bws[k], sems[k])

    def body(i, _):
        for k in range(2):
            ch = 2 * i + k
            b = base0 + ch * CHG
            pltpu.make_async_copy(wmsg_h.at[pl.ds(b, CHG)], bws[k],
                                  sems[k]).wait()
            pltpu.sync_copy(bws[k], shared.at[dvs[k]], add=True)

            @pl.when(ch + 2 < nch)
            def _():
                b2 = base0 + (ch + 2) * CHG
                pltpu.sync_copy(dst_h.at[pl.ds(b2, CHG)], dvs[k])
                pltpu.async_copy(wmsg_h.at[pl.ds(b2, CHG)], bws[k], sems[k])
        return 0

    lax.fori_loop(0, nch // 2, body, 0)
    plsc.subcore_barrier()
    pltpu.sync_copy(shared.at[pl.ds(sid * stripe, stripe)],
                    out.at[pl.ds(cid * NP + sid * stripe, stripe)])


_NBLK = 1000
_EBLK = 2048


def _full(shape):
    return pl.BlockSpec(shape, lambda i: tuple(0 for _ in shape))


def _tc_h0(x, W, b):
    def body(xr, wr, br, out):
        out[...] = jax.nn.relu(
            jnp.dot(xr[...], wr[...], preferred_element_type=f32) + br[...])
    return pl.pallas_call(
        body,
        grid=(N // _NBLK,),
        in_specs=[pl.BlockSpec((_NBLK, H), lambda i: (i, 0)),
                  _full((H, H)), _full((1, H))],
        out_specs=pl.BlockSpec((_NBLK, H), lambda i: (i, 0)),
        out_shape=jax.ShapeDtypeStruct((N, H), f32),
    )(x, W, b)


def _tc_node(h, nt_r, hW, hb, aWd, aWs, lWn):
    def body(hr, ntr, hWr, hbr, aWdr, aWsr, lWnr, stab_o, mnode_o):
        hv = hr[...]
        nt = ntr[0, 0, :].reshape(_NBLK, 1)
        h2 = jnp.zeros((_NBLK, H), f32)
        for t in range(TN):
            proj = jnp.dot(hv, hWr[t], preferred_element_type=f32) \
                + hbr[t, :].reshape(1, H)
            h2 = h2 + jnp.where(nt == t, proj, 0.0)
        sd = jnp.dot(h2, aWdr[...], preferred_element_type=f32)
        ss = jnp.dot(h2, aWsr[...], preferred_element_type=f32)
        stab_o[...] = jnp.concatenate([sd, ss], axis=1)
        mnode_o[...] = jnp.dot(h2, lWnr[...], preferred_element_type=f32)

    return pl.pallas_call(
        body,
        grid=(N // _NBLK,),
        in_specs=[pl.BlockSpec((_NBLK, H), lambda i: (i, 0)),
                  pl.BlockSpec((1, 1, _NBLK), lambda i: (i, 0, 0)),
                  _full((TN, H, H)), _full((TN, H)),
                  _full((H, HEADS)), _full((H, HEADS)), _full((H, H))],
        out_specs=[pl.BlockSpec((_NBLK, 4), lambda i: (i, 0)),
                   pl.BlockSpec((_NBLK, H), lambda i: (i, 0))],
        out_shape=[jax.ShapeDtypeStruct((N, 4), f32),
                   jax.ShapeDtypeStruct((N, H), f32)],
    )(h, nt_r, hW, hb, aWd, aWs, lWn)


def _tc_edgec(et_r, ea, tab, aWe, aWa, eaW):
    def body(etr, ear, tabr, aWer, aWar, eaWr, c_o):
        ctab = jnp.dot(_lky(tabr[...], 0.01), aWer[...],
                       preferred_element_type=f32)
        et = etr[0, 0, :].reshape(_EBLK, 1)
        iot = lax.broadcasted_iota(i32, (_EBLK, TE), 1)
        oh = jnp.where(et == iot, 1.0, 0.0).astype(f32)
        eae = _lky(jnp.dot(ear[...], eaWr[...], preferred_element_type=f32),
                   0.01)
        c_o[...] = jnp.dot(oh, ctab, preferred_element_type=f32) \
            + jnp.dot(eae, aWar[...], preferred_element_type=f32)

    return pl.pallas_call(
        body,
        grid=(EP // _EBLK,),
        in_specs=[pl.BlockSpec((1, 1, _EBLK), lambda i: (i, 0, 0)),
                  pl.BlockSpec((_EBLK, 2), lambda i: (i, 0)),
                  _full((TE, ETE)), _full((ETE, HEADS)),
                  _full((EAE, HEADS)), _full((2, EAE))],
        out_specs=pl.BlockSpec((_EBLK, 2), lambda i: (i, 0)),
        out_shape=jax.ShapeDtypeStruct((EP, 2), f32),
    )(et_r, ea, tab, aWe, aWa, eaW)


def _tc_rtab(dp):
    def body(dr, out):
        a = dr[...]
        out[...] = 0.5 / (a[0] + a[1] + 1e-16)
    return pl.pallas_call(
        body,
        grid=(1,),
        in_specs=[_full((2, NP, 2))],
        out_specs=_full((NP, 2)),
        out_shape=jax.ShapeDtypeStruct((NP, 2), f32),
    )(dp)


def _tc_wmsg(G, w, ea, eaW, W6):
    def body(Gr, wr, ear, eaWr, W6r, out):
        eae = _lky(jnp.dot(ear[...], eaWr[...], preferred_element_type=f32),
                   0.01)
        msg = Gr[...] + jnp.dot(eae, W6r[...], preferred_element_type=f32)
        out[...] = wr[...] * msg

    return pl.pallas_call(
        body,
        grid=(EP // _EBLK,),
        in_specs=[pl.BlockSpec((_EBLK, H), lambda i: (i, 0)),
                  pl.BlockSpec((_EBLK, 1), lambda i: (i, 0)),
                  pl.BlockSpec((_EBLK, 2), lambda i: (i, 0)),
                  _full((2, EAE)), _full((EAE, H))],
        out_specs=pl.BlockSpec((_EBLK, H), lambda i: (i, 0)),
        out_shape=jax.ShapeDtypeStruct((EP, H), f32),
    )(G, w, ea, eaW, W6)


def _tc_hsum(apart):
    def body(ar, out):
        a = ar[...]
        out[...] = a[0] + a[1]
    return pl.pallas_call(
        body,
        grid=(N // _NBLK,),
        in_specs=[pl.BlockSpec((2, _NBLK, H), lambda i: (0, i, 0))],
        out_specs=pl.BlockSpec((_NBLK, H), lambda i: (i, 0)),
        out_shape=jax.ShapeDtypeStruct((N, H), f32),
    )(apart)


def _tc_out(apart, W, b):
    def body(ar, wr, br, out):
        a = ar[...]
        out[...] = jnp.dot(a[0] + a[1], wr[...],
                           preferred_element_type=f32) + br[...]
    return pl.pallas_call(
        body,
        grid=(N // _NBLK,),
        in_specs=[pl.BlockSpec((2, _NBLK, H), lambda i: (0, i, 0)),
                  _full((H, 64)), _full((1, 64))],
        out_specs=pl.BlockSpec((_NBLK, 64), lambda i: (i, 0)),
        out_shape=jax.ShapeDtypeStruct((N, 64), f32),
    )(apart, W, b)


def kernel(x, edge_index, node_type, edge_type, edge_attr, lin_in_W, lin_in_b,
           hetero_W, hetero_b, edge_type_tab, edge_attr_W, att_W, lin_W,
           lin_out_W, lin_out_b):
    pad = EP - E
    src_p = jnp.concatenate([edge_index[0].astype(i32),
                             jnp.zeros((pad,), i32)])
    dst_p = jnp.concatenate([edge_index[1].astype(i32),
                             jnp.full((pad,), N, i32)])
    ea_p = jnp.concatenate([edge_attr, jnp.zeros((pad, 2), f32)])
    et_p = jnp.concatenate([edge_type.astype(i32), jnp.zeros((pad,), i32)])
    nt_r = node_type.astype(i32).reshape(N // _NBLK, 1, _NBLK)
    et_r = et_p.reshape(EP // _EBLK, 1, _EBLK)
    z2 = jnp.zeros((2 * NP,), f32)
    z128 = jnp.zeros((NP, H), f32)
    zstab = jnp.zeros((NP - N, 4), f32)

    h = _tc_h0(x, lin_in_W, lin_in_b.reshape(1, H))
    apart = None
    for l in range(2):
        aW = att_W[l]
        stab, mnode = _tc_node(h, nt_r, hetero_W[l], hetero_b[l],
                               aW[0:H], aW[H:2 * H], lin_W[l][0:H])
        stab_f = jnp.concatenate([stab, zstab]).reshape(4 * NP)
        c = _tc_edgec(et_r, ea_p, edge_type_tab[l],
                      aW[2 * H:2 * H + ETE], aW[2 * H + ETE:],
                      edge_attr_W[l])
        exf, dflat = _sc_attn(stab_f, c.reshape(2 * EP), src_p, dst_p, z2)
        wv, G = _sc_wg(dflat, exf, mnode, src_p, dst_p)
        wmsg = _tc_wmsg(G, wv.reshape(EP, 1), ea_p, edge_attr_W[l],
                        lin_W[l][H:])
        (apart,) = _sc_agg(wmsg, dst_p, z128)
        if l == 0:
            h = _tc_hsum(apart.reshape(2, NP, H))
    return _tc_out(apart.reshape(2, NP, H), lin_out_W, lin_out_b.reshape(1, 64))


# speedup vs baseline: 2.1058x; 1.0420x over previous
import functools

import jax
import jax.numpy as jnp
from jax import lax
from jax.experimental import pallas as pl
from jax.experimental.pallas import tpu as pltpu
from jax.experimental.pallas import tpu_sc as plsc

N = 10000
NP = 10240
E = 160000
EP = 163840
H = 128
HEADS = 2
TN = 5
TE = 5
ETE = 5
EAE = 6
NEG = 0.2

f32 = jnp.float32
i32 = jnp.int32

_info = plsc.get_sparse_core_info()
NC = _info.num_cores
NS = _info.num_subcores
NW = NC * NS
EPW = EP // NW
CHA = 128
CHG = 128
NGA = CHA // 16

_MESH = dict(core_axis_name="c", subcore_axis_name="s")


def _wid():
    return lax.axis_index("s") * NC + lax.axis_index("c")


def _lky(v, slope):
    return jnp.where(v >= 0, v, slope * v)


@functools.partial(
    pl.kernel,
    out_type=[jax.ShapeDtypeStruct((2 * EP,), f32),
              jax.ShapeDtypeStruct((4 * NP,), f32)],
    mesh=plsc.VectorSubcoreMesh(**_MESH),
    compiler_params=pltpu.CompilerParams(needs_layout_passes=False),
    scratch_types=[pltpu.VMEM((CHA,), i32), pltpu.VMEM((CHA,), i32),
                   pltpu.VMEM((2 * CHA,), f32), pltpu.VMEM((2 * CHA,), f32),
                   pltpu.VMEM((CHA,), i32), pltpu.VMEM((CHA,), i32),
                   pltpu.VMEM((4 * NP,), f32),
                   pltpu.VMEM_SHARED((2 * NP,), f32),
                   pltpu.SemaphoreType.DMA],
)
def _sc_attn(stab_h, c_h, src_h, dst_h, z2_h, ex_o, dp_o,
             dv, sv, cbuf, exbuf, idxa, idxb, stab_v, shared, sem):
    cid = lax.axis_index("c")
    sid = lax.axis_index("s")
    stripe = 2 * NP // NS
    pltpu.sync_copy(stab_h, stab_v)
    pltpu.sync_copy(z2_h.at[pl.ds(sid * stripe, stripe)],
                    shared.at[pl.ds(sid * stripe, stripe)])
    plsc.subcore_barrier()
    base0 = _wid() * EPW
    iota = lax.iota(i32, 16)

    def body(i, _):
        b = base0 + i * CHA
        pltpu.async_copy(dst_h.at[pl.ds(b, CHA)], dv, sem)
        pltpu.async_copy(src_h.at[pl.ds(b, CHA)], sv, sem)
        pltpu.async_copy(c_h.at[pl.ds(2 * b, 2 * CHA)], cbuf, sem)
        pltpu.make_async_copy(dst_h.at[pl.ds(b, CHA)], dv, sem).wait()
        pltpu.make_async_copy(src_h.at[pl.ds(b, CHA)], sv, sem).wait()
        pltpu.make_async_copy(c_h.at[pl.ds(2 * b, 2 * CHA)], cbuf, sem).wait()
        for g in range(NGA):
            d16 = dv[pl.ds(g * 16, 16)]
            s16 = sv[pl.ds(g * 16, 16)]
            pos = 2 * (g * 16 + iota)
            sd0 = plsc.load_gather(stab_v, [d16 * 4])
            sd1 = plsc.load_gather(stab_v, [d16 * 4 + 1])
            ss0 = plsc.load_gather(stab_v, [s16 * 4 + 2])
            ss1 = plsc.load_gather(stab_v, [s16 * 4 + 3])
            c0 = plsc.load_gather(cbuf, [pos])
            c1 = plsc.load_gather(cbuf, [pos + 1])
            e0 = jnp.exp(_lky(sd0 + ss0 + c0, NEG))
            e1 = jnp.exp(_lky(sd1 + ss1 + c1, NEG))
            plsc.store_scatter(exbuf, [pos], e0)
            plsc.store_scatter(exbuf, [pos + 1], e1)
            ib = idxa if g < NGA // 2 else idxb
            ip = pos if g < NGA // 2 else pos - CHA
            plsc.store_scatter(ib, [ip], d16 * 2)
            plsc.store_scatter(ib, [ip + 1], d16 * 2 + 1)
        pltpu.sync_copy(exbuf, ex_o.at[pl.ds(2 * b, 2 * CHA)])
        pltpu.sync_copy(exbuf.at[pl.ds(0, CHA)], shared.at[idxa], add=True)
        pltpu.sync_copy(exbuf.at[pl.ds(CHA, CHA)], shared.at[idxb], add=True)
        return 0

    lax.fori_loop(0, EPW // CHA, body, 0)
    plsc.subcore_barrier()
    pltpu.sync_copy(shared.at[pl.ds(sid * stripe, stripe)],
                    dp_o.at[pl.ds(cid * 2 * NP + sid * stripe, stripe)])


@functools.partial(
    pl.kernel,
    out_type=[jax.ShapeDtypeStruct((EP,), f32),
              jax.ShapeDtypeStruct((EP, H), f32)],
    mesh=plsc.VectorSubcoreMesh(**_MESH),
    compiler_params=pltpu.CompilerParams(needs_layout_passes=False),
    scratch_types=[pltpu.VMEM((CHG,), i32), pltpu.VMEM((CHG,), i32),
                   pltpu.VMEM((CHG,), i32),
                   pltpu.VMEM((2 * CHG,), f32), pltpu.VMEM((CHG,), f32),
                   pltpu.VMEM((4 * NP,), f32),
                   pltpu.VMEM((2 * NP,), f32), pltpu.VMEM((CHG, H), f32),
                   pltpu.VMEM((CHG, H), f32),
                   pltpu.SemaphoreType.DMA, pltpu.SemaphoreType.DMA],
)
def _sc_wg(dp_h, ex_h, mnode_h, src_h, dst_h, w_o, g_o,
           sv0, sv1, dv, exl, wbuf, dpv, rtab_v, bg0, bg1, sem0, sem1):
    pltpu.sync_copy(dp_h, dpv)

    def rbody(j, _):
        sl = pl.ds(j * 16, 16)
        sl2 = pl.ds(2 * NP + j * 16, 16)
        rtab_v[sl] = 0.5 / (dpv[sl] + dpv[sl2] + 1e-16)
        return 0

    lax.fori_loop(0, 2 * NP // 16, rbody, 0)
    base0 = _wid() * EPW
    iota = lax.iota(i32, 16)
    nch = EPW // CHG
    svs = [sv0, sv1]
    bgs = [bg0, bg1]
    sems = [sem0, sem1]
    for k in range(2):
        b = base0 + k * CHG
        pltpu.sync_copy(src_h.at[pl.ds(b, CHG)], svs[k])
        pltpu.async_copy(mnode_h.at[svs[k]], bgs[k], sems[k])

    def body(i, _):
        for k in range(2):
            ch = 2 * i + k
            b = base0 + ch * CHG
            pltpu.sync_copy(dst_h.at[pl.ds(b, CHG)], dv)
            pltpu.sync_copy(ex_h.at[pl.ds(2 * b, 2 * CHG)], exl)
            for g in range(CHG // 16):
                d16 = dv[pl.ds(g * 16, 16)]
                pos = 2 * (g * 16 + iota)
                r0 = plsc.load_gather(rtab_v, [d16 * 2])
                r1 = plsc.load_gather(rtab_v, [d16 * 2 + 1])
                e0 = plsc.load_gather(exl, [pos])
                e1 = plsc.load_gather(exl, [pos + 1])
                wbuf[pl.ds(g * 16, 16)] = e0 * r0 + e1 * r1
            pltpu.sync_copy(wbuf, w_o.at[pl.ds(b, CHG)])
            pltpu.make_async_copy(mnode_h.at[svs[k]], bgs[k], sems[k]).wait()
            pltpu.sync_copy(bgs[k], g_o.at[pl.ds(b, CHG)])

            @pl.when(ch + 2 < nch)
            def _():
                b2 = base0 + (ch + 2) * CHG
                pltpu.sync_copy(src_h.at[pl.ds(b2, CHG)], svs[k])
                pltpu.async_copy(mnode_h.at[svs[k]], bgs[k], sems[k])
        return 0

    lax.fori_loop(0, nch // 2, body, 0)


@functools.partial(
    pl.kernel,
    out_type=[jax.ShapeDtypeStruct((2 * NP, H), f32)],
    mesh=plsc.VectorSubcoreMesh(**_MESH),
    compiler_params=pltpu.CompilerParams(needs_layout_passes=False),
    scratch_types=[pltpu.VMEM((CHG,), i32), pltpu.VMEM((CHG,), i32),
                   pltpu.VMEM((CHG, H), f32), pltpu.VMEM((CHG, H), f32),
                   pltpu.VMEM_SHARED((NP, H), f32),
                   pltpu.SemaphoreType.DMA, pltpu.SemaphoreType.DMA],
)
def _sc_agg(wmsg_h, dst_h, z_h, out, dv0, dv1, bw0, bw1, shared,
            sem0, sem1):
    cid = lax.axis_index("c")
    sid = lax.axis_index("s")
    stripe = NP // NS
    nch = EPW // CHG
    dvs = [dv0, dv1]
    bws = [bw0, bw1]
    sems = [sem0, sem1]
    pltpu.sync_copy(z_h.at[pl.ds(sid * stripe, stripe)],
                    shared.at[pl.ds(sid * stripe, stripe)])
    plsc.subcore_barrier()
    base0 = _wid() * EPW
    for k in range(2):
        b = base0 + k * CHG
        pltpu.sync_copy(dst_h.at[pl.ds(b, CHG)], dvs[k])
        pltpu.async_copy(wmsg_h.at[pl.ds(b, CHG)], bws[k], sems[k])

    def body(i, _):
        for k in range(2):
            ch = 2 * i + k
            b = base0 + ch * CHG
            pltpu.make_async_copy(wmsg_h.at[pl.ds(b, CHG)], bws[k],
                                  sems[k]).wait()
            pltpu.sync_copy(bws[k], shared.at[dvs[k]], add=True)

            @pl.when(ch + 2 < nch)
            def _():
                b2 = base0 + (ch + 2) * CHG
                pltpu.sync_copy(dst_h.at[pl.ds(b2, CHG)], dvs[k])
                pltpu.async_copy(wmsg_h.at[pl.ds(b2, CHG)], bws[k], sems[k])
        return 0

    lax.fori_loop(0, nch // 2, body, 0)
    plsc.subcore_barrier()
    pltpu.sync_copy(shared.at[pl.ds(sid * stripe, stripe)],
                    out.at[pl.ds(cid * NP + sid * stripe, stripe)])


_NBLK = 1000
_EBLK = 2048


def _full(shape):
    return pl.BlockSpec(shape, lambda i: tuple(0 for _ in shape))


def _tc_h0(x, W, b):
    def body(xr, wr, br, out):
        out[...] = jax.nn.relu(
            jnp.dot(xr[...], wr[...], preferred_element_type=f32) + br[...])
    return pl.pallas_call(
        body,
        grid=(N // _NBLK,),
        in_specs=[pl.BlockSpec((_NBLK, H), lambda i: (i, 0)),
                  _full((H, H)), _full((1, H))],
        out_specs=pl.BlockSpec((_NBLK, H), lambda i: (i, 0)),
        out_shape=jax.ShapeDtypeStruct((N, H), f32),
    )(x, W, b)


def _tc_node(h, nt_r, hW, hb, aWd, aWs, lWn):
    def body(hr, ntr, hWr, hbr, aWdr, aWsr, lWnr, stab_o, mnode_o):
        hv = hr[...]
        nt = ntr[0, 0, :].reshape(_NBLK, 1)
        h2 = jnp.zeros((_NBLK, H), f32)
        for t in range(TN):
            proj = jnp.dot(hv, hWr[t], preferred_element_type=f32) \
                + hbr[t, :].reshape(1, H)
            h2 = h2 + jnp.where(nt == t, proj, 0.0)
        sd = jnp.dot(h2, aWdr[...], preferred_element_type=f32)
        ss = jnp.dot(h2, aWsr[...], preferred_element_type=f32)
        stab_o[...] = jnp.concatenate([sd, ss], axis=1)
        mnode_o[...] = jnp.dot(h2, lWnr[...], preferred_element_type=f32)

    return pl.pallas_call(
        body,
        grid=(N // _NBLK,),
        in_specs=[pl.BlockSpec((_NBLK, H), lambda i: (i, 0)),
                  pl.BlockSpec((1, 1, _NBLK), lambda i: (i, 0, 0)),
                  _full((TN, H, H)), _full((TN, H)),
                  _full((H, HEADS)), _full((H, HEADS)), _full((H, H))],
        out_specs=[pl.BlockSpec((_NBLK, 4), lambda i: (i, 0)),
                   pl.BlockSpec((_NBLK, H), lambda i: (i, 0))],
        out_shape=[jax.ShapeDtypeStruct((N, 4), f32),
                   jax.ShapeDtypeStruct((N, H), f32)],
    )(h, nt_r, hW, hb, aWd, aWs, lWn)


def _tc_edgec(et_r, ea, tab, aWe, aWa, eaW):
    def body(etr, ear, tabr, aWer, aWar, eaWr, c_o):
        ctab = jnp.dot(_lky(tabr[...], 0.01), aWer[...],
                       preferred_element_type=f32)
        et = etr[0, 0, :].reshape(_EBLK, 1)
        iot = lax.broadcasted_iota(i32, (_EBLK, TE), 1)
        oh = jnp.where(et == iot, 1.0, 0.0).astype(f32)
        eae = _lky(jnp.dot(ear[...], eaWr[...], preferred_element_type=f32),
                   0.01)
        c_o[...] = jnp.dot(oh, ctab, preferred_element_type=f32) \
            + jnp.dot(eae, aWar[...], preferred_element_type=f32)

    return pl.pallas_call(
        body,
        grid=(EP // _EBLK,),
        in_specs=[pl.BlockSpec((1, 1, _EBLK), lambda i: (i, 0, 0)),
                  pl.BlockSpec((_EBLK, 2), lambda i: (i, 0)),
                  _full((TE, ETE)), _full((ETE, HEADS)),
                  _full((EAE, HEADS)), _full((2, EAE))],
        out_specs=pl.BlockSpec((_EBLK, 2), lambda i: (i, 0)),
        out_shape=jax.ShapeDtypeStruct((EP, 2), f32),
    )(et_r, ea, tab, aWe, aWa, eaW)


def _tc_rtab(dp):
    def body(dr, out):
        a = dr[...]
        out[...] = 0.5 / (a[0] + a[1] + 1e-16)
    return pl.pallas_call(
        body,
        grid=(1,),
        in_specs=[_full((2, NP, 2))],
        out_specs=_full((NP, 2)),
        out_shape=jax.ShapeDtypeStruct((NP, 2), f32),
    )(dp)


def _tc_wmsg(G, w, ea, eaW, W6):
    def body(Gr, wr, ear, eaWr, W6r, out):
        eae = _lky(jnp.dot(ear[...], eaWr[...], preferred_element_type=f32),
                   0.01)
        msg = Gr[...] + jnp.dot(eae, W6r[...], preferred_element_type=f32)
        out[...] = wr[...] * msg

    return pl.pallas_call(
        body,
        grid=(EP // _EBLK,),
        in_specs=[pl.BlockSpec((_EBLK, H), lambda i: (i, 0)),
                  pl.BlockSpec((_EBLK, 1), lambda i: (i, 0)),
                  pl.BlockSpec((_EBLK, 2), lambda i: (i, 0)),
                  _full((2, EAE)), _full((EAE, H))],
        out_specs=pl.BlockSpec((_EBLK, H), lambda i: (i, 0)),
        out_shape=jax.ShapeDtypeStruct((EP, H), f32),
    )(G, w, ea, eaW, W6)


def _tc_hsum(apart):
    def body(ar, out):
        a = ar[...]
        out[...] = a[0] + a[1]
    return pl.pallas_call(
        body,
        grid=(N // _NBLK,),
        in_specs=[pl.BlockSpec((2, _NBLK, H), lambda i: (0, i, 0))],
        out_specs=pl.BlockSpec((_NBLK, H), lambda i: (i, 0)),
        out_shape=jax.ShapeDtypeStruct((N, H), f32),
    )(apart)


def _tc_out(apart, W, b):
    def body(ar, wr, br, out):
        a = ar[...]
        out[...] = jnp.dot(a[0] + a[1], wr[...],
                           preferred_element_type=f32) + br[...]
    return pl.pallas_call(
        body,
        grid=(N // _NBLK,),
        in_specs=[pl.BlockSpec((2, _NBLK, H), lambda i: (0, i, 0)),
                  _full((H, 64)), _full((1, 64))],
        out_specs=pl.BlockSpec((_NBLK, 64), lambda i: (i, 0)),
        out_shape=jax.ShapeDtypeStruct((N, 64), f32),
    )(apart, W, b)


def kernel(x, edge_index, node_type, edge_type, edge_attr, lin_in_W, lin_in_b,
           hetero_W, hetero_b, edge_type_tab, edge_attr_W, att_W, lin_W,
           lin_out_W, lin_out_b):
    pad = EP - E
    src_p = jnp.concatenate([edge_index[0].astype(i32),
                             jnp.zeros((pad,), i32)])
    dst_p = jnp.concatenate([edge_index[1].astype(i32),
                             jnp.full((pad,), N, i32)])
    ea_p = jnp.concatenate([edge_attr, jnp.zeros((pad, 2), f32)])
    et_p = jnp.concatenate([edge_type.astype(i32), jnp.zeros((pad,), i32)])
    nt_r = node_type.astype(i32).reshape(N // _NBLK, 1, _NBLK)
    et_r = et_p.reshape(EP // _EBLK, 1, _EBLK)
    z2 = jnp.zeros((2 * NP,), f32)
    z128 = jnp.zeros((NP, H), f32)
    zstab = jnp.zeros((NP - N, 4), f32)

    h = _tc_h0(x, lin_in_W, lin_in_b.reshape(1, H))
    apart = None
    for l in range(2):
        aW = att_W[l]
        stab, mnode = _tc_node(h, nt_r, hetero_W[l], hetero_b[l],
                               aW[0:H], aW[H:2 * H], lin_W[l][0:H])
        stab_f = jnp.concatenate([stab, zstab]).reshape(4 * NP)
        c = _tc_edgec(et_r, ea_p, edge_type_tab[l],
                      aW[2 * H:2 * H + ETE], aW[2 * H + ETE:],
                      edge_attr_W[l])
        exf, dflat = _sc_attn(stab_f, c.reshape(2 * EP), src_p, dst_p, z2)
        wv, G = _sc_wg(dflat, exf, mnode, src_p, dst_p)
        wmsg = _tc_wmsg(G, wv.reshape(EP, 1), ea_p, edge_attr_W[l],
                        lin_W[l][H:])
        (apart,) = _sc_agg(wmsg, dst_p, z128)
        if l == 0:
            h = _tc_hsum(apart.reshape(2, NP, H))
    return _tc_out(apart.reshape(2, NP, H), lin_out_W, lin_out_b.reshape(1, 64))


# speedup vs baseline: 2.1088x; 1.0014x over previous
import functools

import jax
import jax.numpy as jnp
from jax import lax
from jax.experimental import pallas as pl
from jax.experimental.pallas import tpu as pltpu
from jax.experimental.pallas import tpu_sc as plsc

N = 10000
NP = 10240
E = 160000
EP = 163840
H = 128
HEADS = 2
TN = 5
TE = 5
ETE = 5
EAE = 6
NEG = 0.2

f32 = jnp.float32
i32 = jnp.int32

_info = plsc.get_sparse_core_info()
NC = _info.num_cores
NS = _info.num_subcores
NW = NC * NS
EPW = EP // NW
CHA = 128
CHG = 128
NGA = CHA // 16

_MESH = dict(core_axis_name="c", subcore_axis_name="s")


def _wid():
    return lax.axis_index("s") * NC + lax.axis_index("c")


def _lky(v, slope):
    return jnp.where(v >= 0, v, slope * v)


@functools.partial(
    pl.kernel,
    out_type=[jax.ShapeDtypeStruct((2 * EP,), f32),
              jax.ShapeDtypeStruct((4 * NP,), f32)],
    mesh=plsc.VectorSubcoreMesh(**_MESH),
    compiler_params=pltpu.CompilerParams(needs_layout_passes=False),
    scratch_types=[pltpu.VMEM((CHA,), i32), pltpu.VMEM((CHA,), i32),
                   pltpu.VMEM((2 * CHA,), f32), pltpu.VMEM((2 * CHA,), f32),
                   pltpu.VMEM((CHA,), i32), pltpu.VMEM((CHA,), i32),
                   pltpu.VMEM((4 * NP,), f32),
                   pltpu.VMEM_SHARED((2 * NP,), f32),
                   pltpu.SemaphoreType.DMA],
)
def _sc_attn(stab_h, c_h, src_h, dst_h, z2_h, ex_o, dp_o,
             dv, sv, cbuf, exbuf, idxa, idxb, stab_v, shared, sem):
    cid = lax.axis_index("c")
    sid = lax.axis_index("s")
    stripe = 2 * NP // NS
    pltpu.sync_copy(stab_h, stab_v)
    pltpu.sync_copy(z2_h.at[pl.ds(sid * stripe, stripe)],
                    shared.at[pl.ds(sid * stripe, stripe)])
    plsc.subcore_barrier()
    base0 = _wid() * EPW
    iota = lax.iota(i32, 16)

    def body(i, _):
        b = base0 + i * CHA
        pltpu.async_copy(dst_h.at[pl.ds(b, CHA)], dv, sem)
        pltpu.async_copy(src_h.at[pl.ds(b, CHA)], sv, sem)
        pltpu.async_copy(c_h.at[pl.ds(2 * b, 2 * CHA)], cbuf, sem)
        pltpu.make_async_copy(dst_h.at[pl.ds(b, CHA)], dv, sem).wait()
        pltpu.make_async_copy(src_h.at[pl.ds(b, CHA)], sv, sem).wait()
        pltpu.make_async_copy(c_h.at[pl.ds(2 * b, 2 * CHA)], cbuf, sem).wait()
        for g in range(NGA):
            d16 = dv[pl.ds(g * 16, 16)]
            s16 = sv[pl.ds(g * 16, 16)]
            pos = 2 * (g * 16 + iota)
            sd0 = plsc.load_gather(stab_v, [d16 * 4])
            sd1 = plsc.load_gather(stab_v, [d16 * 4 + 1])
            ss0 = plsc.load_gather(stab_v, [s16 * 4 + 2])
            ss1 = plsc.load_gather(stab_v, [s16 * 4 + 3])
            c0 = plsc.load_gather(cbuf, [pos])
            c1 = plsc.load_gather(cbuf, [pos + 1])
            e0 = jnp.exp(_lky(sd0 + ss0 + c0, NEG))
            e1 = jnp.exp(_lky(sd1 + ss1 + c1, NEG))
            plsc.store_scatter(exbuf, [pos], e0)
            plsc.store_scatter(exbuf, [pos + 1], e1)
            ib = idxa if g < NGA // 2 else idxb
            ip = pos if g < NGA // 2 else pos - CHA
            plsc.store_scatter(ib, [ip], d16 * 2)
            plsc.store_scatter(ib, [ip + 1], d16 * 2 + 1)
        pltpu.sync_copy(exbuf, ex_o.at[pl.ds(2 * b, 2 * CHA)])
        pltpu.sync_copy(exbuf.at[pl.ds(0, CHA)], shared.at[idxa], add=True)
        pltpu.sync_copy(exbuf.at[pl.ds(CHA, CHA)], shared.at[idxb], add=True)
        return 0

    lax.fori_loop(0, EPW // CHA, body, 0)
    plsc.subcore_barrier()
    pltpu.sync_copy(shared.at[pl.ds(sid * stripe, stripe)],
                    dp_o.at[pl.ds(cid * 2 * NP + sid * stripe, stripe)])


@functools.partial(
    pl.kernel,
    out_type=[jax.ShapeDtypeStruct((EP,), f32),
              jax.ShapeDtypeStruct((EP, H), f32)],
    mesh=plsc.VectorSubcoreMesh(**_MESH),
    compiler_params=pltpu.CompilerParams(needs_layout_passes=False),
    scratch_types=[pltpu.VMEM((CHG,), i32), pltpu.VMEM((CHG,), i32),
                   pltpu.VMEM((CHG,), i32),
                   pltpu.VMEM((2 * CHG,), f32), pltpu.VMEM((CHG,), f32),
                   pltpu.VMEM((4 * NP,), f32),
                   pltpu.VMEM((2 * NP,), f32), pltpu.VMEM((CHG, H), f32),
                   pltpu.VMEM((CHG, H), f32),
                   pltpu.SemaphoreType.DMA, pltpu.SemaphoreType.DMA],
)
def _sc_wg(dp_h, ex_h, mnode_h, src_h, dst_h, w_o, g_o,
           sv0, sv1, dv, exl, wbuf, dpv, rtab_v, bg0, bg1, sem0, sem1):
    pltpu.sync_copy(dp_h, dpv)

    def rbody(j, _):
        sl = pl.ds(j * 16, 16)
        sl2 = pl.ds(2 * NP + j * 16, 16)
        rtab_v[sl] = 0.5 / (dpv[sl] + dpv[sl2] + 1e-16)
        return 0

    lax.fori_loop(0, 2 * NP // 16, rbody, 0)
    base0 = _wid() * EPW
    iota = lax.iota(i32, 16)
    nch = EPW // CHG
    svs = [sv0, sv1]
    bgs = [bg0, bg1]
    sems = [sem0, sem1]
    for k in range(2):
        b = base0 + k * CHG
        pltpu.sync_copy(src_h.at[pl.ds(b, CHG)], svs[k])
        pltpu.async_copy(mnode_h.at[svs[k]], bgs[k], sems[k])

    def body(i, _):
        for k in range(2):
            ch = 2 * i + k
            b = base0 + ch * CHG
            pltpu.sync_copy(dst_h.at[pl.ds(b, CHG)], dv)
            pltpu.sync_copy(ex_h.at[pl.ds(2 * b, 2 * CHG)], exl)
            for g in range(CHG // 16):
                d16 = dv[pl.ds(g * 16, 16)]
                pos = 2 * (g * 16 + iota)
                r0 = plsc.load_gather(rtab_v, [d16 * 2])
                r1 = plsc.load_gather(rtab_v, [d16 * 2 + 1])
                e0 = plsc.load_gather(exl, [pos])
                e1 = plsc.load_gather(exl, [pos + 1])
                wbuf[pl.ds(g * 16, 16)] = e0 * r0 + e1 * r1
            pltpu.sync_copy(wbuf, w_o.at[pl.ds(b, CHG)])
            pltpu.make_async_copy(mnode_h.at[svs[k]], bgs[k], sems[k]).wait()
            pltpu.sync_copy(bgs[k], g_o.at[pl.ds(b, CHG)])

            @pl.when(ch + 2 < nch)
            def _():
                b2 = base0 + (ch + 2) * CHG
                pltpu.sync_copy(src_h.at[pl.ds(b2, CHG)], svs[k])
                pltpu.async_copy(mnode_h.at[svs[k]], bgs[k], sems[k])
        return 0

    lax.fori_loop(0, nch // 2, body, 0)


@functools.partial(
    pl.kernel,
    out_type=[jax.ShapeDtypeStruct((2 * NP, H), f32)],
    mesh=plsc.VectorSubcoreMesh(**_MESH),
    compiler_params=pltpu.CompilerParams(needs_layout_passes=False),
    scratch_types=[pltpu.VMEM((CHG,), i32), pltpu.VMEM((CHG,), i32),
                   pltpu.VMEM((CHG, H), f32), pltpu.VMEM((CHG, H), f32),
                   pltpu.VMEM_SHARED((NP, H), f32),
                   pltpu.SemaphoreType.DMA, pltpu.SemaphoreType.DMA],
)
def _sc_agg(wmsg_h, dst_h, z_h, out, dv0, dv1, bw0, bw1, shared,
            sem0, sem1):
    cid = lax.axis_index("c")
    sid = lax.axis_index("s")
    stripe = NP // NS
    nch = EPW // CHG
    dvs = [dv0, dv1]
    bws = [bw0, bw1]
    sems = [sem0, sem1]
    pltpu.sync_copy(z_h.at[pl.ds(sid * stripe, stripe)],
                    shared.at[pl.ds(sid * stripe, stripe)])
    plsc.subcore_barrier()
    base0 = _wid() * EPW
    for k in range(2):
        b = base0 + k * CHG
        pltpu.sync_copy(dst_h.at[pl.ds(b, CHG)], dvs[k])
        pltpu.async_copy(wmsg_h.at[pl.ds(b, CHG)], bws[k], sems[k])

    def body(i, _):
        for k in range(2):
            ch = 2 * i + k
            b = base0 + ch * CHG
            pltpu.make_async_copy(wmsg_h.at[pl.ds(b, CHG)], bws[k],
                                  sems[k]).wait()
            pltpu.sync_copy(bws[k], shared.at[dvs[k]], add=True)

            @pl.when(ch + 2 < nch)
            def _():
                b2 = base0 + (ch + 2) * CHG
                pltpu.sync_copy(dst_h.at[pl.ds(b2, CHG)], dvs[k])
                pltpu.async_copy(wmsg_h.at[pl.ds(b2, CHG)], bws[k], sems[k])
        return 0

    lax.fori_loop(0, nch // 2, body, 0)
    plsc.subcore_barrier()
    pltpu.sync_copy(shared.at[pl.ds(sid * stripe, stripe)],
                    out.at[pl.ds(cid * NP + sid * stripe, stripe)])


_NBLK = 1000
_EBLK = 2048


def _full(shape):
    return pl.BlockSpec(shape, lambda i: tuple(0 for _ in shape))


def _tc_h0(x, W, b):
    def body(xr, wr, br, out):
        out[...] = jax.nn.relu(
            jnp.dot(xr[...], wr[...], preferred_element_type=f32) + br[...])
    return pl.pallas_call(
        body,
        grid=(N // _NBLK,),
        in_specs=[pl.BlockSpec((_NBLK, H), lambda i: (i, 0)),
                  _full((H, H)), _full((1, H))],
        out_specs=pl.BlockSpec((_NBLK, H), lambda i: (i, 0)),
        out_shape=jax.ShapeDtypeStruct((N, H), f32),
    )(x, W, b)


def _tc_node(h, nt_r, hW, hb, aWd, aWs, lWn):
    def body(hr, ntr, hWr, hbr, aWdr, aWsr, lWnr, stab_o, mnode_o):
        hv = hr[...]
        nt = ntr[0, 0, :].reshape(_NBLK, 1)
        h2 = jnp.zeros((_NBLK, H), f32)
        for t in range(TN):
            proj = jnp.dot(hv, hWr[t], preferred_element_type=f32) \
                + hbr[t, :].reshape(1, H)
            h2 = h2 + jnp.where(nt == t, proj, 0.0)
        sd = jnp.dot(h2, aWdr[...], preferred_element_type=f32)
        ss = jnp.dot(h2, aWsr[...], preferred_element_type=f32)
        stab_o[...] = jnp.concatenate([sd, ss], axis=1)
        mnode_o[...] = jnp.dot(h2, lWnr[...], preferred_element_type=f32)

    return pl.pallas_call(
        body,
        grid=(N // _NBLK,),
        in_specs=[pl.BlockSpec((_NBLK, H), lambda i: (i, 0)),
                  pl.BlockSpec((1, 1, _NBLK), lambda i: (i, 0, 0)),
                  _full((TN, H, H)), _full((TN, H)),
                  _full((H, HEADS)), _full((H, HEADS)), _full((H, H))],
        out_specs=[pl.BlockSpec((_NBLK, 4), lambda i: (i, 0)),
                   pl.BlockSpec((_NBLK, H), lambda i: (i, 0))],
        out_shape=[jax.ShapeDtypeStruct((N, 4), f32),
                   jax.ShapeDtypeStruct((N, H), f32)],
    )(h, nt_r, hW, hb, aWd, aWs, lWn)


def _tc_edgec(et_r, ea, tab, aWe, aWa, eaW):
    def body(etr, ear, tabr, aWer, aWar, eaWr, c_o):
        ctab = jnp.dot(_lky(tabr[...], 0.01), aWer[...],
                       preferred_element_type=f32)
        et = etr[0, 0, :].reshape(_EBLK, 1)
        iot = lax.broadcasted_iota(i32, (_EBLK, TE), 1)
        oh = jnp.where(et == iot, 1.0, 0.0).astype(f32)
        eae = _lky(jnp.dot(ear[...], eaWr[...], preferred_element_type=f32),
                   0.01)
        c_o[...] = jnp.dot(oh, ctab, preferred_element_type=f32) \
            + jnp.dot(eae, aWar[...], preferred_element_type=f32)

    return pl.pallas_call(
        body,
        grid=(EP // _EBLK,),
        in_specs=[pl.BlockSpec((1, 1, _EBLK), lambda i: (i, 0, 0)),
                  pl.BlockSpec((_EBLK, 2), lambda i: (i, 0)),
                  _full((TE, ETE)), _full((ETE, HEADS)),
                  _full((EAE, HEADS)), _full((2, EAE))],
        out_specs=pl.BlockSpec((_EBLK, 2), lambda i: (i, 0)),
        out_shape=jax.ShapeDtypeStruct((EP, 2), f32),
    )(et_r, ea, tab, aWe, aWa, eaW)


def _tc_wmsg(G, w, ea, eaW, W6):
    def body(Gr, wr, ear, eaWr, W6r, out):
        eae = _lky(jnp.dot(ear[...], eaWr[...], preferred_element_type=f32),
                   0.01)
        msg = Gr[...] + jnp.dot(eae, W6r[...], preferred_element_type=f32)
        out[...] = wr[...] * msg

    return pl.pallas_call(
        body,
        grid=(EP // _EBLK,),
        in_specs=[pl.BlockSpec((_EBLK, H), lambda i: (i, 0)),
                  pl.BlockSpec((_EBLK, 1), lambda i: (i, 0)),
                  pl.BlockSpec((_EBLK, 2), lambda i: (i, 0)),
                  _full((2, EAE)), _full((EAE, H))],
        out_specs=pl.BlockSpec((_EBLK, H), lambda i: (i, 0)),
        out_shape=jax.ShapeDtypeStruct((EP, H), f32),
    )(G, w, ea, eaW, W6)


def _tc_hsum(apart):
    def body(ar, out):
        a = ar[...]
        out[...] = a[0] + a[1]
    return pl.pallas_call(
        body,
        grid=(N // _NBLK,),
        in_specs=[pl.BlockSpec((2, _NBLK, H), lambda i: (0, i, 0))],
        out_specs=pl.BlockSpec((_NBLK, H), lambda i: (i, 0)),
        out_shape=jax.ShapeDtypeStruct((N, H), f32),
    )(apart)


def _tc_out(apart, W, b):
    def body(ar, wr, br, out):
        a = ar[...]
        out[...] = jnp.dot(a[0] + a[1], wr[...],
                           preferred_element_type=f32) + br[...]
    return pl.pallas_call(
        body,
        grid=(N // _NBLK,),
        in_specs=[pl.BlockSpec((2, _NBLK, H), lambda i: (0, i, 0)),
                  _full((H, 64)), _full((1, 64))],
        out_specs=pl.BlockSpec((_NBLK, 64), lambda i: (i, 0)),
        out_shape=jax.ShapeDtypeStruct((N, 64), f32),
    )(apart, W, b)


def kernel(x, edge_index, node_type, edge_type, edge_attr, lin_in_W, lin_in_b,
           hetero_W, hetero_b, edge_type_tab, edge_attr_W, att_W, lin_W,
           lin_out_W, lin_out_b):
    pad = EP - E
    src_p = jnp.concatenate([edge_index[0].astype(i32),
                             jnp.zeros((pad,), i32)])
    dst_p = jnp.concatenate([edge_index[1].astype(i32),
                             jnp.full((pad,), N, i32)])
    ea_p = jnp.concatenate([edge_attr, jnp.zeros((pad, 2), f32)])
    et_p = jnp.concatenate([edge_type.astype(i32), jnp.zeros((pad,), i32)])
    nt_r = node_type.astype(i32).reshape(N // _NBLK, 1, _NBLK)
    et_r = et_p.reshape(EP // _EBLK, 1, _EBLK)
    z2 = jnp.zeros((2 * NP,), f32)
    z128 = jnp.zeros((NP, H), f32)
    zstab = jnp.zeros((NP - N, 4), f32)

    h = _tc_h0(x, lin_in_W, lin_in_b.reshape(1, H))
    apart = None
    for l in range(2):
        aW = att_W[l]
        stab, mnode = _tc_node(h, nt_r, hetero_W[l], hetero_b[l],
                               aW[0:H], aW[H:2 * H], lin_W[l][0:H])
        stab_f = jnp.concatenate([stab, zstab]).reshape(4 * NP)
        c = _tc_edgec(et_r, ea_p, edge_type_tab[l],
                      aW[2 * H:2 * H + ETE], aW[2 * H + ETE:],
                      edge_attr_W[l])
        exf, dflat = _sc_attn(stab_f, c.reshape(2 * EP), src_p, dst_p, z2)
        wv, G = _sc_wg(dflat, exf, mnode, src_p, dst_p)
        wmsg = _tc_wmsg(G, wv.reshape(EP, 1), ea_p, edge_attr_W[l],
                        lin_W[l][H:])
        (apart,) = _sc_agg(wmsg, dst_p, z128)
        if l == 0:
            h = _tc_hsum(apart.reshape(2, NP, H))
    return _tc_out(apart.reshape(2, NP, H), lin_out_W, lin_out_b.reshape(1, 64))
